# Initial kernel scaffold; baseline (speedup 1.0000x reference)
import jax
import jax.numpy as jnp
from jax.experimental import pallas as pl


def kernel(x, edge_index, edge_type, W_rel1, W_self1, b1, W_rel2, W_self2, b2, inter_logits, gate_log_alpha, absent_bias):
    raise NotImplementedError("write your pallas kernel here")


# speedup vs baseline: 7.4589x; 7.4589x over previous
import functools

import jax
import jax.numpy as jnp
from jax import lax
from jax.experimental import pallas as pl
from jax.experimental.pallas import tpu as pltpu
from jax.experimental.pallas import tpu_sc as plsc

N = 10000
E = 320000
R = 4
DIN = 128
DH = 128
C = 64
TEMP = 2.0 / 3.0
STRETCH_EPS = 0.1
CLIP = 2.0

NC = 2
NS = 16
L = 16

K = 80
ER = E // K
EPT = E // (NC * NS)
EPS_SC = E // NS
CH = 2000
DROWS = 512

_f32 = jnp.float32
_i32 = jnp.int32

_mesh = plsc.VectorSubcoreMesh(core_axis_name="c", subcore_axis_name="s")


BN = 2000


def _enc1_body(x_ref, wr_ref, ws_ref, b_ref, y_ref, s_ref):
    xb = x_ref[...]
    for r in range(R):
        y_ref[r] = jnp.dot(xb, wr_ref[r], preferred_element_type=_f32)
    s_ref[...] = jnp.dot(xb, ws_ref[...], preferred_element_type=_f32) + b_ref[...]


def _enc1(x, W_rel1, W_self1, b1):
    return pl.pallas_call(
        _enc1_body,
        grid=(N // BN,),
        in_specs=[
            pl.BlockSpec((BN, DIN), lambda i: (i, 0)),
            pl.BlockSpec((R, DIN, DH), lambda i: (0, 0, 0)),
            pl.BlockSpec((DIN, DH), lambda i: (0, 0)),
            pl.BlockSpec((1, DH), lambda i: (0, 0)),
        ],
        out_specs=[
            pl.BlockSpec((R, BN, DH), lambda i: (0, i, 0)),
            pl.BlockSpec((BN, DH), lambda i: (i, 0)),
        ],
        out_shape=[
            jax.ShapeDtypeStruct((R, N, DH), _f32),
            jax.ShapeDtypeStruct((N, DH), _f32),
        ],
    )(x, W_rel1, W_self1, b1)


def _enc2_body(s1_ref, m_ref, wr_ref, ws_ref, b_ref, y_ref, s_ref):
    h = jnp.maximum(s1_ref[...] + m_ref[0] + m_ref[1], 0.0)
    pad = jnp.zeros((BN, DH - C), _f32)
    for r in range(R):
        yr = jnp.dot(h, wr_ref[r], preferred_element_type=_f32)
        y_ref[r] = jnp.concatenate([yr, pad], axis=1)
    s_ref[...] = jnp.dot(h, ws_ref[...], preferred_element_type=_f32) + b_ref[...]


def _enc2(S1, M1, W_rel2, W_self2, b2):
    return pl.pallas_call(
        _enc2_body,
        grid=(N // BN,),
        in_specs=[
            pl.BlockSpec((BN, DH), lambda i: (i, 0)),
            pl.BlockSpec((2, BN, DH), lambda i: (0, i, 0)),
            pl.BlockSpec((R, DH, C), lambda i: (0, 0, 0)),
            pl.BlockSpec((DH, C), lambda i: (0, 0)),
            pl.BlockSpec((1, C), lambda i: (0, 0)),
        ],
        out_specs=[
            pl.BlockSpec((R, BN, DH), lambda i: (0, i, 0)),
            pl.BlockSpec((BN, C), lambda i: (i, 0)),
        ],
        out_shape=[
            jax.ShapeDtypeStruct((R, N, DH), _f32),
            jax.ShapeDtypeStruct((N, C), _f32),
        ],
    )(S1, M1, W_rel2, W_self2, b2)


def _enc3_body(s2_ref, m_ref, il_ref, ga_ref, ab_ref, a_ref, t_ref):
    logits = s2_ref[...] + m_ref[0][:, :C] + m_ref[1][:, :C]
    a = jax.nn.softmax(logits, axis=-1)
    pad = jnp.zeros((BN, DH - C), _f32)
    a_ref[...] = jnp.concatenate([a, pad], axis=1)
    pre = jnp.clip(ga_ref[...] / TEMP, -CLIP, CLIP)
    z = jax.nn.sigmoid(pre) * (1.0 + 2.0 * STRETCH_EPS) - STRETCH_EPS
    gate = jnp.clip(z, 0.0, 1.0)
    w = jax.nn.sigmoid(il_ref[...]) * gate
    abv = ab_ref[...]
    for r in range(R):
        tr = (jnp.dot(a, w[r], preferred_element_type=_f32)
              + abv[0:1, r:r + 1])
        t_ref[r] = jnp.concatenate([tr, pad], axis=1)


def _enc3(S2, M2, inter_logits, gate_log_alpha, ab):
    return pl.pallas_call(
        _enc3_body,
        grid=(N // BN,),
        in_specs=[
            pl.BlockSpec((BN, C), lambda i: (i, 0)),
            pl.BlockSpec((2, BN, DH), lambda i: (0, i, 0)),
            pl.BlockSpec((R, C, C), lambda i: (0, 0, 0)),
            pl.BlockSpec((R, C, C), lambda i: (0, 0, 0)),
            pl.BlockSpec((1, R), lambda i: (0, 0)),
        ],
        out_specs=[
            pl.BlockSpec((BN, DH), lambda i: (i, 0)),
            pl.BlockSpec((R, BN, DH), lambda i: (0, i, 0)),
        ],
        out_shape=[
            jax.ShapeDtypeStruct((N, DH), _f32),
            jax.ShapeDtypeStruct((R, N, DH), _f32),
        ],
    )(S2, M2, inter_logits, gate_log_alpha, ab)


@functools.partial(
    pl.kernel,
    out_type=(
        jax.ShapeDtypeStruct((E,), _i32),
        jax.ShapeDtypeStruct((E,), _f32),
    ),
    mesh=_mesh,
    compiler_params=pltpu.CompilerParams(needs_layout_passes=False),
    scratch_types=[
        pltpu.VMEM((DROWS, 128), _f32),
        pltpu.VMEM((CH,), _i32),
        pltpu.VMEM((CH,), _i32),
        pltpu.VMEM((CH,), _i32),
        pltpu.VMEM((32, 128), _i32),
        pltpu.VMEM((CH,), _i32),
        pltpu.VMEM((CH,), _f32),
        pltpu.VMEM_SHARED((DROWS, 128), _f32),
    ],
)
def _prep(src_hbm, dst_hbm, et_hbm, gsrc_hbm, sedge_hbm,
          deg_t, srcb, dstb, etb, idxrows, gout, sout, deg_s):
    cid = lax.axis_index("c")
    sid = lax.axis_index("s")
    zero16 = jnp.zeros((L,), _f32)

    def _z(j, _):
        for q in range(128 // L):
            deg_t[j, pl.ds(q * L, L)] = zero16
        return _
    lax.fori_loop(0, DROWS, _z, None)

    @pl.when(sid == 0)
    def _():
        pltpu.sync_copy(deg_t, deg_s)
    plsc.subcore_barrier()

    def _hist_chunk(ci, _):
        base = sid * EPS_SC + ci * CH
        pltpu.sync_copy(dst_hbm.at[pl.ds(base, CH)], dstb)
        pltpu.sync_copy(et_hbm.at[pl.ds(base, CH)], etb)

        def _grp(g, _):
            d16 = dstb[pl.ds(g * L, L)]
            e16 = etb[pl.ds(g * L, L)]
            idx = e16 * N + d16
            cnt, lastm = plsc.scan_count(idx)
            row = lax.shift_right_logical(idx, 7)
            col = jnp.bitwise_and(idx, 127)
            plsc.addupdate_scatter(deg_t, [row, col], cnt.astype(_f32),
                                   mask=lastm)
            return _
        lax.fori_loop(0, CH // L, _grp, None)
        return _
    lax.fori_loop(0, EPS_SC // CH, _hist_chunk, None)
    plsc.subcore_barrier()

    iota16 = lax.iota(_i32, L)
    for cc in range(4):
        for gg in range(8):
            idxrows[cc * 8, pl.ds(gg * L, L)] = iota16 + (cc * 128 + gg * L)
    for cc in range(4):
        pltpu.sync_copy(deg_t.at[pl.ds(cc * 128, 128)],
                        deg_s.at[idxrows.at[cc * 8]], add=True)
    plsc.subcore_barrier()

    pltpu.sync_copy(deg_s, deg_t)

    def _s(j, _):
        for q in range(128 // L):
            v = deg_t[j, pl.ds(q * L, L)]
            deg_t[j, pl.ds(q * L, L)] = 1.0 / jnp.maximum(v, 1.0)
        return _
    lax.fori_loop(0, DROWS, _s, None)

    gwid = cid * NS + sid

    def _out_chunk(ci, _):
        base = gwid * EPT + ci * CH
        pltpu.sync_copy(src_hbm.at[pl.ds(base, CH)], srcb)
        pltpu.sync_copy(dst_hbm.at[pl.ds(base, CH)], dstb)
        pltpu.sync_copy(et_hbm.at[pl.ds(base, CH)], etb)

        def _grp(g, _):
            s16 = srcb[pl.ds(g * L, L)]
            d16 = dstb[pl.ds(g * L, L)]
            e16 = etb[pl.ds(g * L, L)]
            gout[pl.ds(g * L, L)] = e16 * N + s16
            sidx = e16 * N + d16
            srow = lax.shift_right_logical(sidx, 7)
            scol = jnp.bitwise_and(sidx, 127)
            sout[pl.ds(g * L, L)] = plsc.load_gather(deg_t, [srow, scol])
            return _
        lax.fori_loop(0, CH // L, _grp, None)
        pltpu.sync_copy(gout, gsrc_hbm.at[pl.ds(base, CH)])
        pltpu.sync_copy(sout, sedge_hbm.at[pl.ds(base, CH)])
        return _
    lax.fori_loop(0, EPT // CH, _out_chunk, None)


def _make_layer(D):
    UNITS = N // L

    @functools.partial(
        pl.kernel,
        out_type=jax.ShapeDtypeStruct((2 * N, D), _f32),
        mesh=_mesh,
        compiler_params=pltpu.CompilerParams(needs_layout_passes=False),
        scratch_types=[
            pltpu.VMEM((CH,), _i32),
            pltpu.VMEM((CH,), _i32),
            pltpu.VMEM((CH,), _f32),
            pltpu.VMEM((K,), _i32),
            pltpu.VMEM((K,), _i32),
            pltpu.VMEM((K, D), _f32),
            pltpu.VMEM((L, D), _f32),
            pltpu.SemaphoreType.DMA,
            pltpu.VMEM_SHARED((N, D), _f32),
        ],
    )
    def _layer(y_hbm, gsrc_hbm, dst_hbm, sed_hbm, m_hbm,
               gbuf, dbuf, sbuf, g80, d80, rows, zb, sem, msh):
        cid = lax.axis_index("c")
        sid = lax.axis_index("s")
        zero16 = jnp.zeros((L,), _f32)

        for j in range(L):
            for d in range(D // L):
                zb[j, pl.ds(d * L, L)] = zero16
        nu = (UNITS - sid + NS - 1) // NS

        def _z(k, _):
            u = sid + k * NS
            pltpu.sync_copy(zb, msh.at[pl.ds(u * L, L)])
            return _
        lax.fori_loop(0, nu, _z, None)
        plsc.subcore_barrier()

        ebase = cid * (E // 2) + sid * EPT

        def _blk(t, _):
            base = ebase + t * CH
            pltpu.sync_copy(gsrc_hbm.at[pl.ds(base, CH)], gbuf)
            pltpu.sync_copy(dst_hbm.at[pl.ds(base, CH)], dbuf)
            pltpu.sync_copy(sed_hbm.at[pl.ds(base, CH)], sbuf)

            def _chunk(i, _):
                for q in range(K // L):
                    g80[pl.ds(q * L, L)] = gbuf[pl.ds(i * K + q * L, L)]
                    d80[pl.ds(q * L, L)] = dbuf[pl.ds(i * K + q * L, L)]
                pltpu.async_copy(y_hbm.at[g80], rows, sem).wait()

                def _scale(g, _):
                    s16 = sbuf[pl.ds(i * K + g * L, L)]
                    for jj in range(L):
                        sv = jnp.full((L,), s16[jj], _f32)
                        j = g * L + jj
                        for d in range(D // L):
                            rows[j, pl.ds(d * L, L)] = (
                                rows[j, pl.ds(d * L, L)] * sv)
                    return _
                lax.fori_loop(0, K // L, _scale, None)
                pltpu.sync_copy(rows, msh.at[d80], add=True)
                return _
            lax.fori_loop(0, CH // K, _chunk, None)
            return _
        lax.fori_loop(0, EPT // CH, _blk, None)
        plsc.subcore_barrier()

        def _wb(k, _):
            off = (sid + k * NS) * L
            pltpu.sync_copy(msh.at[pl.ds(off, L)],
                            m_hbm.at[pl.ds(cid * N + off, L)])
            return _
        lax.fori_loop(0, nu, _wb, None)

    return _layer


_layer128 = _make_layer(DH)


@functools.partial(
    pl.kernel,
    out_type=jax.ShapeDtypeStruct((E,), _f32),
    mesh=_mesh,
    compiler_params=pltpu.CompilerParams(needs_layout_passes=False),
    scratch_types=[
        pltpu.VMEM((CH,), _i32),
        pltpu.VMEM((CH,), _i32),
        pltpu.VMEM((K,), _i32),
        pltpu.VMEM((K,), _i32),
        pltpu.VMEM((K, DH), _f32),
        pltpu.VMEM((K, DH), _f32),
        pltpu.VMEM((CH,), _f32),
        pltpu.SemaphoreType.DMA,
        pltpu.SemaphoreType.DMA,
    ],
)
def _final(t_hbm, a_hbm, gsrc_hbm, dst_hbm, out_hbm,
           gbuf, dbuf, g80, d80, tr, ar, ob, sem1, sem2):
    cid = lax.axis_index("c")
    sid = lax.axis_index("s")
    gwid = cid * NS + sid
    ebase = gwid * EPT

    def _blk(t, _):
        base = ebase + t * CH
        pltpu.sync_copy(gsrc_hbm.at[pl.ds(base, CH)], gbuf)
        pltpu.sync_copy(dst_hbm.at[pl.ds(base, CH)], dbuf)

        def _chunk(i, _):
            for q in range(K // L):
                g80[pl.ds(q * L, L)] = gbuf[pl.ds(i * K + q * L, L)]
                d80[pl.ds(q * L, L)] = dbuf[pl.ds(i * K + q * L, L)]
            cp1 = pltpu.async_copy(t_hbm.at[g80], tr, sem1)
            cp2 = pltpu.async_copy(a_hbm.at[d80], ar, sem2)
            cp1.wait()
            cp2.wait()

            def _dot(g, _):
                r16 = g * L + lax.iota(_i32, L)

                def _d(d, acc):
                    cd = jnp.full((L,), d, _i32)
                    gt = plsc.load_gather(tr, [r16, cd])
                    ga = plsc.load_gather(ar, [r16, cd])
                    return acc + gt * ga
                acc = lax.fori_loop(0, C, _d, jnp.zeros((L,), _f32))
                ob[pl.ds(i * K + g * L, L)] = acc
                return _
            lax.fori_loop(0, K // L, _dot, None)
            return _
        lax.fori_loop(0, CH // K, _chunk, None)
        pltpu.sync_copy(ob, out_hbm.at[pl.ds(base, CH)])
        return _
    lax.fori_loop(0, EPT // CH, _blk, None)


def kernel(x, edge_index, edge_type, W_rel1, W_self1, b1, W_rel2, W_self2,
           b2, inter_logits, gate_log_alpha, absent_bias):
    src = edge_index[0].astype(_i32)
    dst = edge_index[1].astype(_i32)
    et = edge_type.astype(_i32)

    Y1, S1 = _enc1(x, W_rel1, W_self1, b1.reshape(1, DH))
    gsrc, sedge = _prep(src, dst, et)

    M1 = _layer128(Y1.reshape(R * N, DH), gsrc, dst, sedge)
    Y2, S2 = _enc2(S1, M1.reshape(2, N, DH), W_rel2, W_self2, b2.reshape(1, C))

    M2 = _layer128(Y2.reshape(R * N, DH), gsrc, dst, sedge)
    A, T = _enc3(S2, M2.reshape(2, N, DH), inter_logits, gate_log_alpha,
                 absent_bias.reshape(1, R))

    return _final(T.reshape(R * N, DH), A, gsrc, dst)


# speedup vs baseline: 10.4623x; 1.4027x over previous
import functools

import jax
import jax.numpy as jnp
from jax import lax
from jax.experimental import pallas as pl
from jax.experimental.pallas import tpu as pltpu
from jax.experimental.pallas import tpu_sc as plsc

N = 10000
E = 320000
R = 4
DIN = 128
DH = 128
C = 64
TEMP = 2.0 / 3.0
STRETCH_EPS = 0.1
CLIP = 2.0

NC = 2
NS = 16
L = 16

K = 80
ER = E // K
EPT = E // (NC * NS)
EPS_SC = E // NS
CH = 2000
DROWS = 512

_f32 = jnp.float32
_i32 = jnp.int32

_mesh = plsc.VectorSubcoreMesh(core_axis_name="c", subcore_axis_name="s")


BN = 2000


def _enc1_body(x_ref, wr_ref, ws_ref, b_ref, y_ref, s_ref):
    xb = x_ref[...]
    for r in range(R):
        y_ref[r] = jnp.dot(xb, wr_ref[r], preferred_element_type=_f32)
    s_ref[...] = jnp.dot(xb, ws_ref[...], preferred_element_type=_f32) + b_ref[...]


def _enc1(x, W_rel1, W_self1, b1):
    return pl.pallas_call(
        _enc1_body,
        grid=(N // BN,),
        in_specs=[
            pl.BlockSpec((BN, DIN), lambda i: (i, 0)),
            pl.BlockSpec((R, DIN, DH), lambda i: (0, 0, 0)),
            pl.BlockSpec((DIN, DH), lambda i: (0, 0)),
            pl.BlockSpec((1, DH), lambda i: (0, 0)),
        ],
        out_specs=[
            pl.BlockSpec((R, BN, DH), lambda i: (0, i, 0)),
            pl.BlockSpec((BN, DH), lambda i: (i, 0)),
        ],
        out_shape=[
            jax.ShapeDtypeStruct((R, N, DH), _f32),
            jax.ShapeDtypeStruct((N, DH), _f32),
        ],
    )(x, W_rel1, W_self1, b1)


def _enc2_body(s1_ref, m_ref, wr_ref, ws_ref, b_ref, y_ref, s_ref):
    h = jnp.maximum(s1_ref[...] + m_ref[0] + m_ref[1], 0.0)
    pad = jnp.zeros((BN, DH - C), _f32)
    for r in range(R):
        yr = jnp.dot(h, wr_ref[r], preferred_element_type=_f32)
        y_ref[r] = jnp.concatenate([yr, pad], axis=1)
    s_ref[...] = jnp.dot(h, ws_ref[...], preferred_element_type=_f32) + b_ref[...]


def _enc2(S1, M1, W_rel2, W_self2, b2):
    return pl.pallas_call(
        _enc2_body,
        grid=(N // BN,),
        in_specs=[
            pl.BlockSpec((BN, DH), lambda i: (i, 0)),
            pl.BlockSpec((2, BN, DH), lambda i: (0, i, 0)),
            pl.BlockSpec((R, DH, C), lambda i: (0, 0, 0)),
            pl.BlockSpec((DH, C), lambda i: (0, 0)),
            pl.BlockSpec((1, C), lambda i: (0, 0)),
        ],
        out_specs=[
            pl.BlockSpec((R, BN, DH), lambda i: (0, i, 0)),
            pl.BlockSpec((BN, C), lambda i: (i, 0)),
        ],
        out_shape=[
            jax.ShapeDtypeStruct((R, N, DH), _f32),
            jax.ShapeDtypeStruct((N, C), _f32),
        ],
    )(S1, M1, W_rel2, W_self2, b2)


def _enc3_body(s2_ref, m_ref, il_ref, ga_ref, ab_ref, a_ref, t_ref):
    logits = s2_ref[...] + m_ref[0][:, :C] + m_ref[1][:, :C]
    a = jax.nn.softmax(logits, axis=-1)
    pad = jnp.zeros((BN, DH - C), _f32)
    a_ref[...] = jnp.concatenate([a, pad], axis=1)
    pre = jnp.clip(ga_ref[...] / TEMP, -CLIP, CLIP)
    z = jax.nn.sigmoid(pre) * (1.0 + 2.0 * STRETCH_EPS) - STRETCH_EPS
    gate = jnp.clip(z, 0.0, 1.0)
    w = jax.nn.sigmoid(il_ref[...]) * gate
    abv = ab_ref[...]
    for r in range(R):
        tr = (jnp.dot(a, w[r], preferred_element_type=_f32)
              + abv[0:1, r:r + 1])
        t_ref[r] = jnp.concatenate([tr, pad], axis=1)


def _enc3(S2, M2, inter_logits, gate_log_alpha, ab):
    return pl.pallas_call(
        _enc3_body,
        grid=(N // BN,),
        in_specs=[
            pl.BlockSpec((BN, C), lambda i: (i, 0)),
            pl.BlockSpec((2, BN, DH), lambda i: (0, i, 0)),
            pl.BlockSpec((R, C, C), lambda i: (0, 0, 0)),
            pl.BlockSpec((R, C, C), lambda i: (0, 0, 0)),
            pl.BlockSpec((1, R), lambda i: (0, 0)),
        ],
        out_specs=[
            pl.BlockSpec((BN, DH), lambda i: (i, 0)),
            pl.BlockSpec((R, BN, DH), lambda i: (0, i, 0)),
        ],
        out_shape=[
            jax.ShapeDtypeStruct((N, DH), _f32),
            jax.ShapeDtypeStruct((R, N, DH), _f32),
        ],
    )(S2, M2, inter_logits, gate_log_alpha, ab)


@functools.partial(
    pl.kernel,
    out_type=(
        jax.ShapeDtypeStruct((E,), _i32),
        jax.ShapeDtypeStruct((E,), _f32),
    ),
    mesh=_mesh,
    compiler_params=pltpu.CompilerParams(needs_layout_passes=False),
    scratch_types=[
        pltpu.VMEM((DROWS, 128), _f32),
        pltpu.VMEM((CH,), _i32),
        pltpu.VMEM((CH,), _i32),
        pltpu.VMEM((CH,), _i32),
        pltpu.VMEM((32, 128), _i32),
        pltpu.VMEM((CH,), _i32),
        pltpu.VMEM((CH,), _f32),
        pltpu.VMEM_SHARED((DROWS, 128), _f32),
    ],
)
def _prep(src_hbm, dst_hbm, et_hbm, gsrc_hbm, sedge_hbm,
          deg_t, srcb, dstb, etb, idxrows, gout, sout, deg_s):
    cid = lax.axis_index("c")
    sid = lax.axis_index("s")
    zero16 = jnp.zeros((L,), _f32)

    def _z(j, _):
        for q in range(128 // L):
            deg_t[j, pl.ds(q * L, L)] = zero16
        return _
    lax.fori_loop(0, DROWS, _z, None)

    @pl.when(sid == 0)
    def _():
        pltpu.sync_copy(deg_t, deg_s)
    plsc.subcore_barrier()

    def _hist_chunk(ci, _):
        base = sid * EPS_SC + ci * CH
        pltpu.sync_copy(dst_hbm.at[pl.ds(base, CH)], dstb)
        pltpu.sync_copy(et_hbm.at[pl.ds(base, CH)], etb)

        def _grp(g, _):
            d16 = dstb[pl.ds(g * L, L)]
            e16 = etb[pl.ds(g * L, L)]
            idx = e16 * N + d16
            cnt, lastm = plsc.scan_count(idx)
            row = lax.shift_right_logical(idx, 7)
            col = jnp.bitwise_and(idx, 127)
            plsc.addupdate_scatter(deg_t, [row, col], cnt.astype(_f32),
                                   mask=lastm)
            return _
        lax.fori_loop(0, CH // L, _grp, None)
        return _
    lax.fori_loop(0, EPS_SC // CH, _hist_chunk, None)
    plsc.subcore_barrier()

    iota16 = lax.iota(_i32, L)
    for cc in range(4):
        for gg in range(8):
            idxrows[cc * 8, pl.ds(gg * L, L)] = iota16 + (cc * 128 + gg * L)
    for cc in range(4):
        pltpu.sync_copy(deg_t.at[pl.ds(cc * 128, 128)],
                        deg_s.at[idxrows.at[cc * 8]], add=True)
    plsc.subcore_barrier()

    pltpu.sync_copy(deg_s, deg_t)

    def _s(j, _):
        for q in range(128 // L):
            v = deg_t[j, pl.ds(q * L, L)]
            deg_t[j, pl.ds(q * L, L)] = 1.0 / jnp.maximum(v, 1.0)
        return _
    lax.fori_loop(0, DROWS, _s, None)

    gwid = cid * NS + sid

    def _out_chunk(ci, _):
        base = gwid * EPT + ci * CH
        pltpu.sync_copy(src_hbm.at[pl.ds(base, CH)], srcb)
        pltpu.sync_copy(dst_hbm.at[pl.ds(base, CH)], dstb)
        pltpu.sync_copy(et_hbm.at[pl.ds(base, CH)], etb)

        def _grp(g, _):
            s16 = srcb[pl.ds(g * L, L)]
            d16 = dstb[pl.ds(g * L, L)]
            e16 = etb[pl.ds(g * L, L)]
            gout[pl.ds(g * L, L)] = e16 * N + s16
            sidx = e16 * N + d16
            srow = lax.shift_right_logical(sidx, 7)
            scol = jnp.bitwise_and(sidx, 127)
            sout[pl.ds(g * L, L)] = plsc.load_gather(deg_t, [srow, scol])
            return _
        lax.fori_loop(0, CH // L, _grp, None)
        pltpu.sync_copy(gout, gsrc_hbm.at[pl.ds(base, CH)])
        pltpu.sync_copy(sout, sedge_hbm.at[pl.ds(base, CH)])
        return _
    lax.fori_loop(0, EPT // CH, _out_chunk, None)


def _make_layer(D):
    RCH = N // K
    NCH = CH // K

    @functools.partial(
        pl.kernel,
        out_type=jax.ShapeDtypeStruct((2 * N, D), _f32),
        mesh=_mesh,
        compiler_params=pltpu.CompilerParams(needs_layout_passes=False),
        scratch_types=[
            pltpu.VMEM((CH,), _i32),
            pltpu.VMEM((CH,), _i32),
            pltpu.VMEM((CH,), _f32),
            pltpu.VMEM((K,), _i32),
            pltpu.VMEM((K,), _i32),
            pltpu.VMEM((K,), _i32),
            pltpu.VMEM((K,), _i32),
            pltpu.VMEM((K, D), _f32),
            pltpu.VMEM((K, D), _f32),
            pltpu.SemaphoreType.DMA,
            pltpu.SemaphoreType.DMA,
            pltpu.VMEM_SHARED((N, D), _f32),
        ],
    )
    def _layer(y_hbm, gsrc_hbm, dst_hbm, sed_hbm, m_hbm,
               gbuf, dbuf, sbuf, g80a, g80b, d80a, d80b, rowsa, rowsb,
               sema, semb, msh):
        cid = lax.axis_index("c")
        sid = lax.axis_index("s")
        zero16 = jnp.zeros((L,), _f32)

        for j in range(K):
            for d in range(D // L):
                rowsa[j, pl.ds(d * L, L)] = zero16
        nu = (RCH - sid + NS - 1) // NS

        def _z(k, _):
            u = sid + k * NS
            pltpu.sync_copy(rowsa, msh.at[pl.ds(u * K, K)])
            return _
        lax.fori_loop(0, nu, _z, None)
        plsc.subcore_barrier()

        ebase = cid * (E // 2) + sid * EPT

        def _fire(i, g80, d80, rows, sem):
            for q in range(K // L):
                g80[pl.ds(q * L, L)] = gbuf[pl.ds(i * K + q * L, L)]
                d80[pl.ds(q * L, L)] = dbuf[pl.ds(i * K + q * L, L)]
            pltpu.async_copy(y_hbm.at[g80], rows, sem)

        def _process(i, g80, d80, rows, sem):
            pltpu.make_async_copy(y_hbm.at[g80], rows, sem).wait()

            def _scale(g, _):
                s16 = sbuf[pl.ds(i * K + g * L, L)]
                for jj in range(L):
                    sv = jnp.full((L,), s16[jj], _f32)
                    j = g * L + jj
                    for d in range(D // L):
                        rows[j, pl.ds(d * L, L)] = rows[j, pl.ds(d * L, L)] * sv
                return _
            lax.fori_loop(0, K // L, _scale, None)
            pltpu.sync_copy(rows, msh.at[d80], add=True)

        def _blk(t, _):
            base = ebase + t * CH
            pltpu.sync_copy(gsrc_hbm.at[pl.ds(base, CH)], gbuf)
            pltpu.sync_copy(dst_hbm.at[pl.ds(base, CH)], dbuf)
            pltpu.sync_copy(sed_hbm.at[pl.ds(base, CH)], sbuf)
            _fire(0, g80a, d80a, rowsa, sema)
            _fire(1, g80b, d80b, rowsb, semb)

            def _pair(pp, _):
                ia = 2 * pp
                _process(ia, g80a, d80a, rowsa, sema)

                @pl.when(ia + 2 < NCH)
                def _():
                    _fire(ia + 2, g80a, d80a, rowsa, sema)

                @pl.when(ia + 1 < NCH)
                def _():
                    _process(ia + 1, g80b, d80b, rowsb, semb)

                    @pl.when(ia + 3 < NCH)
                    def _():
                        _fire(ia + 3, g80b, d80b, rowsb, semb)
                return _
            lax.fori_loop(0, (NCH + 1) // 2, _pair, None)
            return _
        lax.fori_loop(0, EPT // CH, _blk, None)
        plsc.subcore_barrier()

        def _wb(k, _):
            off = (sid + k * NS) * K
            pltpu.sync_copy(msh.at[pl.ds(off, K)],
                            m_hbm.at[pl.ds(cid * N + off, K)])
            return _
        lax.fori_loop(0, nu, _wb, None)

    return _layer


_layer128 = _make_layer(DH)


@functools.partial(
    pl.kernel,
    out_type=jax.ShapeDtypeStruct((E,), _f32),
    mesh=_mesh,
    compiler_params=pltpu.CompilerParams(needs_layout_passes=False),
    scratch_types=[
        pltpu.VMEM((EPT,), _i32),
        pltpu.VMEM((EPT,), _i32),
        pltpu.VMEM((K,), _i32),
        pltpu.VMEM((K,), _i32),
        pltpu.VMEM((K,), _i32),
        pltpu.VMEM((K,), _i32),
        pltpu.VMEM((K, DH), _f32),
        pltpu.VMEM((K, DH), _f32),
        pltpu.VMEM((K, DH), _f32),
        pltpu.VMEM((K, DH), _f32),
        pltpu.VMEM((EPT,), _f32),
        pltpu.SemaphoreType.DMA,
        pltpu.SemaphoreType.DMA,
        pltpu.SemaphoreType.DMA,
        pltpu.SemaphoreType.DMA,
    ],
)
def _final(t_hbm, a_hbm, gsrc_hbm, dst_hbm, out_hbm,
           gbuf, dbuf, g80a, g80b, d80a, d80b, tra, trb, ara, arb, ob,
           semta, semtb, semaa, semab):
    cid = lax.axis_index("c")
    sid = lax.axis_index("s")
    gwid = cid * NS + sid
    ebase = gwid * EPT
    NCH = EPT // K

    pltpu.sync_copy(gsrc_hbm.at[pl.ds(ebase, EPT)], gbuf)
    pltpu.sync_copy(dst_hbm.at[pl.ds(ebase, EPT)], dbuf)

    def _fire(i, g80, d80, tr, ar, semt, sema):
        for q in range(K // L):
            g80[pl.ds(q * L, L)] = gbuf[pl.ds(i * K + q * L, L)]
            d80[pl.ds(q * L, L)] = dbuf[pl.ds(i * K + q * L, L)]
        pltpu.async_copy(t_hbm.at[g80], tr, semt)
        pltpu.async_copy(a_hbm.at[d80], ar, sema)

    def _process(i, g80, d80, tr, ar, semt, sema):
        pltpu.make_async_copy(t_hbm.at[g80], tr, semt).wait()
        pltpu.make_async_copy(a_hbm.at[d80], ar, sema).wait()

        def _dot(g, _):
            r16 = g * L + lax.iota(_i32, L)

            def _d(d, acc):
                cd = jnp.full((L,), d, _i32)
                gt = plsc.load_gather(tr, [r16, cd])
                ga = plsc.load_gather(ar, [r16, cd])
                return acc + gt * ga
            acc = lax.fori_loop(0, C, _d, jnp.zeros((L,), _f32))
            ob[pl.ds(i * K + g * L, L)] = acc
            return _
        lax.fori_loop(0, K // L, _dot, None)

    _fire(0, g80a, d80a, tra, ara, semta, semaa)
    _fire(1, g80b, d80b, trb, arb, semtb, semab)

    def _pair(pp, _):
        ia = 2 * pp
        _process(ia, g80a, d80a, tra, ara, semta, semaa)

        @pl.when(ia + 2 < NCH)
        def _():
            _fire(ia + 2, g80a, d80a, tra, ara, semta, semaa)

        @pl.when(ia + 1 < NCH)
        def _():
            _process(ia + 1, g80b, d80b, trb, arb, semtb, semab)

            @pl.when(ia + 3 < NCH)
            def _():
                _fire(ia + 3, g80b, d80b, trb, arb, semtb, semab)
        return _
    lax.fori_loop(0, (NCH + 1) // 2, _pair, None)

    pltpu.sync_copy(ob, out_hbm.at[pl.ds(ebase, EPT)])


def kernel(x, edge_index, edge_type, W_rel1, W_self1, b1, W_rel2, W_self2,
           b2, inter_logits, gate_log_alpha, absent_bias):
    src = edge_index[0].astype(_i32)
    dst = edge_index[1].astype(_i32)
    et = edge_type.astype(_i32)

    Y1, S1 = _enc1(x, W_rel1, W_self1, b1.reshape(1, DH))
    gsrc, sedge = _prep(src, dst, et)

    M1 = _layer128(Y1.reshape(R * N, DH), gsrc, dst, sedge)
    Y2, S2 = _enc2(S1, M1.reshape(2, N, DH), W_rel2, W_self2, b2.reshape(1, C))

    M2 = _layer128(Y2.reshape(R * N, DH), gsrc, dst, sedge)
    A, T = _enc3(S2, M2.reshape(2, N, DH), inter_logits, gate_log_alpha,
                 absent_bias.reshape(1, R))

    return _final(T.reshape(R * N, DH), A, gsrc, dst)


# speedup vs baseline: 10.5024x; 1.0038x over previous
import functools

import jax
import jax.numpy as jnp
from jax import lax
from jax.experimental import pallas as pl
from jax.experimental.pallas import tpu as pltpu
from jax.experimental.pallas import tpu_sc as plsc

N = 10000
E = 320000
R = 4
DIN = 128
DH = 128
C = 64
TEMP = 2.0 / 3.0
STRETCH_EPS = 0.1
CLIP = 2.0

NC = 2
NS = 16
L = 16

K = 80
ER = E // K
EPT = E // (NC * NS)
EPS_SC = E // NS
CH = 2000
DROWS = 512

_f32 = jnp.float32
_i32 = jnp.int32

_mesh = plsc.VectorSubcoreMesh(core_axis_name="c", subcore_axis_name="s")


BN = 2000


def _enc1_body(x_ref, wr_ref, ws_ref, b_ref, y_ref, s_ref):
    xb = x_ref[...]
    for r in range(R):
        y_ref[r] = jnp.dot(xb, wr_ref[r], preferred_element_type=_f32)
    s_ref[...] = jnp.dot(xb, ws_ref[...], preferred_element_type=_f32) + b_ref[...]


def _enc1(x, W_rel1, W_self1, b1):
    return pl.pallas_call(
        _enc1_body,
        grid=(N // BN,),
        in_specs=[
            pl.BlockSpec((BN, DIN), lambda i: (i, 0)),
            pl.BlockSpec((R, DIN, DH), lambda i: (0, 0, 0)),
            pl.BlockSpec((DIN, DH), lambda i: (0, 0)),
            pl.BlockSpec((1, DH), lambda i: (0, 0)),
        ],
        out_specs=[
            pl.BlockSpec((R, BN, DH), lambda i: (0, i, 0)),
            pl.BlockSpec((BN, DH), lambda i: (i, 0)),
        ],
        out_shape=[
            jax.ShapeDtypeStruct((R, N, DH), _f32),
            jax.ShapeDtypeStruct((N, DH), _f32),
        ],
    )(x, W_rel1, W_self1, b1)


def _enc2_body(s1_ref, m_ref, wr_ref, ws_ref, b_ref, y_ref, s_ref):
    h = jnp.maximum(s1_ref[...] + m_ref[0] + m_ref[1], 0.0)
    pad = jnp.zeros((BN, DH - C), _f32)
    for r in range(R):
        yr = jnp.dot(h, wr_ref[r], preferred_element_type=_f32)
        y_ref[r] = jnp.concatenate([yr, pad], axis=1)
    s_ref[...] = jnp.dot(h, ws_ref[...], preferred_element_type=_f32) + b_ref[...]


def _enc2(S1, M1, W_rel2, W_self2, b2):
    return pl.pallas_call(
        _enc2_body,
        grid=(N // BN,),
        in_specs=[
            pl.BlockSpec((BN, DH), lambda i: (i, 0)),
            pl.BlockSpec((2, BN, DH), lambda i: (0, i, 0)),
            pl.BlockSpec((R, DH, C), lambda i: (0, 0, 0)),
            pl.BlockSpec((DH, C), lambda i: (0, 0)),
            pl.BlockSpec((1, C), lambda i: (0, 0)),
        ],
        out_specs=[
            pl.BlockSpec((R, BN, DH), lambda i: (0, i, 0)),
            pl.BlockSpec((BN, C), lambda i: (i, 0)),
        ],
        out_shape=[
            jax.ShapeDtypeStruct((R, N, DH), _f32),
            jax.ShapeDtypeStruct((N, C), _f32),
        ],
    )(S1, M1, W_rel2, W_self2, b2)


def _enc3_body(s2_ref, m_ref, il_ref, ga_ref, ab_ref, a_ref, t_ref):
    logits = s2_ref[...] + m_ref[0][:, :C] + m_ref[1][:, :C]
    a = jax.nn.softmax(logits, axis=-1)
    pad = jnp.zeros((BN, DH - C), _f32)
    a_ref[...] = jnp.concatenate([a, pad], axis=1)
    pre = jnp.clip(ga_ref[...] / TEMP, -CLIP, CLIP)
    z = jax.nn.sigmoid(pre) * (1.0 + 2.0 * STRETCH_EPS) - STRETCH_EPS
    gate = jnp.clip(z, 0.0, 1.0)
    w = jax.nn.sigmoid(il_ref[...]) * gate
    abv = ab_ref[...]
    for r in range(R):
        tr = (jnp.dot(a, w[r], preferred_element_type=_f32)
              + abv[0:1, r:r + 1])
        t_ref[r] = jnp.concatenate([tr, pad], axis=1)


def _enc3(S2, M2, inter_logits, gate_log_alpha, ab):
    return pl.pallas_call(
        _enc3_body,
        grid=(N // BN,),
        in_specs=[
            pl.BlockSpec((BN, C), lambda i: (i, 0)),
            pl.BlockSpec((2, BN, DH), lambda i: (0, i, 0)),
            pl.BlockSpec((R, C, C), lambda i: (0, 0, 0)),
            pl.BlockSpec((R, C, C), lambda i: (0, 0, 0)),
            pl.BlockSpec((1, R), lambda i: (0, 0)),
        ],
        out_specs=[
            pl.BlockSpec((BN, DH), lambda i: (i, 0)),
            pl.BlockSpec((R, BN, DH), lambda i: (0, i, 0)),
        ],
        out_shape=[
            jax.ShapeDtypeStruct((N, DH), _f32),
            jax.ShapeDtypeStruct((R, N, DH), _f32),
        ],
    )(S2, M2, inter_logits, gate_log_alpha, ab)


@functools.partial(
    pl.kernel,
    out_type=(
        jax.ShapeDtypeStruct((E,), _i32),
        jax.ShapeDtypeStruct((E,), _f32),
    ),
    mesh=_mesh,
    compiler_params=pltpu.CompilerParams(needs_layout_passes=False),
    scratch_types=[
        pltpu.VMEM((DROWS, 128), _f32),
        pltpu.VMEM((CH,), _i32),
        pltpu.VMEM((CH,), _i32),
        pltpu.VMEM((CH,), _i32),
        pltpu.VMEM((32, 128), _i32),
        pltpu.VMEM((CH,), _i32),
        pltpu.VMEM((CH,), _f32),
        pltpu.VMEM_SHARED((DROWS, 128), _f32),
    ],
)
def _prep(src_hbm, dst_hbm, et_hbm, gsrc_hbm, sedge_hbm,
          deg_t, srcb, dstb, etb, idxrows, gout, sout, deg_s):
    cid = lax.axis_index("c")
    sid = lax.axis_index("s")
    zero16 = jnp.zeros((L,), _f32)

    def _z(j, _):
        for q in range(128 // L):
            deg_t[j, pl.ds(q * L, L)] = zero16
        return _
    lax.fori_loop(0, DROWS, _z, None)

    @pl.when(sid == 0)
    def _():
        pltpu.sync_copy(deg_t, deg_s)
    plsc.subcore_barrier()

    def _hist_chunk(ci, _):
        base = sid * EPS_SC + ci * CH
        pltpu.sync_copy(dst_hbm.at[pl.ds(base, CH)], dstb)
        pltpu.sync_copy(et_hbm.at[pl.ds(base, CH)], etb)

        def _grp(g, _):
            d16 = dstb[pl.ds(g * L, L)]
            e16 = etb[pl.ds(g * L, L)]
            idx = e16 * N + d16
            cnt, lastm = plsc.scan_count(idx)
            row = lax.shift_right_logical(idx, 7)
            col = jnp.bitwise_and(idx, 127)
            plsc.addupdate_scatter(deg_t, [row, col], cnt.astype(_f32),
                                   mask=lastm)
            return _
        lax.fori_loop(0, CH // L, _grp, None)
        return _
    lax.fori_loop(0, EPS_SC // CH, _hist_chunk, None)
    plsc.subcore_barrier()

    iota16 = lax.iota(_i32, L)
    for cc in range(4):
        for gg in range(8):
            idxrows[cc * 8, pl.ds(gg * L, L)] = iota16 + (cc * 128 + gg * L)
    for cc in range(4):
        pltpu.sync_copy(deg_t.at[pl.ds(cc * 128, 128)],
                        deg_s.at[idxrows.at[cc * 8]], add=True)
    plsc.subcore_barrier()

    pltpu.sync_copy(deg_s, deg_t)

    def _s(j, _):
        for q in range(128 // L):
            v = deg_t[j, pl.ds(q * L, L)]
            deg_t[j, pl.ds(q * L, L)] = 1.0 / jnp.maximum(v, 1.0)
        return _
    lax.fori_loop(0, DROWS, _s, None)

    gwid = cid * NS + sid

    def _out_chunk(ci, _):
        base = gwid * EPT + ci * CH
        pltpu.sync_copy(src_hbm.at[pl.ds(base, CH)], srcb)
        pltpu.sync_copy(dst_hbm.at[pl.ds(base, CH)], dstb)
        pltpu.sync_copy(et_hbm.at[pl.ds(base, CH)], etb)

        def _grp(g, _):
            s16 = srcb[pl.ds(g * L, L)]
            d16 = dstb[pl.ds(g * L, L)]
            e16 = etb[pl.ds(g * L, L)]
            gout[pl.ds(g * L, L)] = e16 * N + s16
            sidx = e16 * N + d16
            srow = lax.shift_right_logical(sidx, 7)
            scol = jnp.bitwise_and(sidx, 127)
            sout[pl.ds(g * L, L)] = plsc.load_gather(deg_t, [srow, scol])
            return _
        lax.fori_loop(0, CH // L, _grp, None)
        pltpu.sync_copy(gout, gsrc_hbm.at[pl.ds(base, CH)])
        pltpu.sync_copy(sout, sedge_hbm.at[pl.ds(base, CH)])
        return _
    lax.fori_loop(0, EPT // CH, _out_chunk, None)


def _make_layer(D):
    RCH = N // K
    NCH = CH // K
    NB = 3

    @functools.partial(
        pl.kernel,
        out_type=jax.ShapeDtypeStruct((2 * N, D), _f32),
        mesh=_mesh,
        compiler_params=pltpu.CompilerParams(needs_layout_passes=False),
        scratch_types=[
            pltpu.VMEM((CH,), _i32),
            pltpu.VMEM((CH,), _i32),
            pltpu.VMEM((CH,), _f32),
            [pltpu.VMEM((K,), _i32)] * NB,
            [pltpu.VMEM((K,), _i32)] * NB,
            [pltpu.VMEM((K, D), _f32)] * NB,
            [pltpu.SemaphoreType.DMA] * NB,
            [pltpu.SemaphoreType.DMA] * NB,
            pltpu.VMEM_SHARED((N, D), _f32),
        ],
    )
    def _layer(y_hbm, gsrc_hbm, dst_hbm, sed_hbm, m_hbm,
               gbuf, dbuf, sbuf, g80s, d80s, rowss, gsems, ssems, msh):
        cid = lax.axis_index("c")
        sid = lax.axis_index("s")
        zero16 = jnp.zeros((L,), _f32)

        rows0 = rowss[0]
        for j in range(K):
            for d in range(D // L):
                rows0[j, pl.ds(d * L, L)] = zero16
        nu = (RCH - sid + NS - 1) // NS

        def _z(k, _):
            u = sid + k * NS
            pltpu.sync_copy(rows0, msh.at[pl.ds(u * K, K)])
            return _
        lax.fori_loop(0, nu, _z, None)
        plsc.subcore_barrier()

        ebase = cid * (E // 2) + sid * EPT

        def _fire(i, sl):
            g80, d80 = g80s[sl], d80s[sl]
            for q in range(K // L):
                g80[pl.ds(q * L, L)] = gbuf[pl.ds(i * K + q * L, L)]
                d80[pl.ds(q * L, L)] = dbuf[pl.ds(i * K + q * L, L)]
            pltpu.async_copy(y_hbm.at[g80], rowss[sl], gsems[sl])

        def _process(i, sl):
            rows = rowss[sl]
            pltpu.make_async_copy(y_hbm.at[g80s[sl]], rows, gsems[sl]).wait()

            def _scale(g, _):
                s16 = sbuf[pl.ds(i * K + g * L, L)]
                for jj in range(L):
                    sv = jnp.full((L,), s16[jj], _f32)
                    j = g * L + jj
                    for d in range(D // L):
                        rows[j, pl.ds(d * L, L)] = rows[j, pl.ds(d * L, L)] * sv
                return _
            lax.fori_loop(0, K // L, _scale, None)
            pltpu.async_copy(rows, msh.at[d80s[sl]], ssems[sl], add=True)

        def _wait_scat(sl):
            pltpu.make_async_copy(rowss[sl], msh.at[d80s[sl]],
                                  ssems[sl]).wait()

        def _blk(t, _):
            base = ebase + t * CH
            pltpu.sync_copy(gsrc_hbm.at[pl.ds(base, CH)], gbuf)
            pltpu.sync_copy(dst_hbm.at[pl.ds(base, CH)], dbuf)
            pltpu.sync_copy(sed_hbm.at[pl.ds(base, CH)], sbuf)
            for sl in range(NB):
                _fire(sl, sl)

            def _trip(pp, _):
                i0 = NB * pp
                for off in range(NB):
                    i = i0 + off

                    @pl.when(i < NCH)
                    def _():
                        _process(i, off)
                for off in range(NB):
                    j = i0 + NB + off

                    @pl.when(j < NCH)
                    def _():
                        _wait_scat(off)
                        _fire(j, off)
                return _
            lax.fori_loop(0, (NCH + NB - 1) // NB, _trip, None)
            for sl in range(NB):
                _wait_scat(sl)
            return _
        lax.fori_loop(0, EPT // CH, _blk, None)
        plsc.subcore_barrier()

        def _wb(k, _):
            off = (sid + k * NS) * K
            pltpu.sync_copy(msh.at[pl.ds(off, K)],
                            m_hbm.at[pl.ds(cid * N + off, K)])
            return _
        lax.fori_loop(0, nu, _wb, None)

    return _layer


_layer128 = _make_layer(DH)


@functools.partial(
    pl.kernel,
    out_type=jax.ShapeDtypeStruct((E,), _f32),
    mesh=_mesh,
    compiler_params=pltpu.CompilerParams(needs_layout_passes=False),
    scratch_types=[
        pltpu.VMEM((EPT,), _i32),
        pltpu.VMEM((EPT,), _i32),
        pltpu.VMEM((K,), _i32),
        pltpu.VMEM((K,), _i32),
        pltpu.VMEM((K,), _i32),
        pltpu.VMEM((K,), _i32),
        pltpu.VMEM((K, DH), _f32),
        pltpu.VMEM((K, DH), _f32),
        pltpu.VMEM((K, DH), _f32),
        pltpu.VMEM((K, DH), _f32),
        pltpu.VMEM((EPT,), _f32),
        pltpu.SemaphoreType.DMA,
        pltpu.SemaphoreType.DMA,
        pltpu.SemaphoreType.DMA,
        pltpu.SemaphoreType.DMA,
    ],
)
def _final(t_hbm, a_hbm, gsrc_hbm, dst_hbm, out_hbm,
           gbuf, dbuf, g80a, g80b, d80a, d80b, tra, trb, ara, arb, ob,
           semta, semtb, semaa, semab):
    cid = lax.axis_index("c")
    sid = lax.axis_index("s")
    gwid = cid * NS + sid
    ebase = gwid * EPT
    NCH = EPT // K

    pltpu.sync_copy(gsrc_hbm.at[pl.ds(ebase, EPT)], gbuf)
    pltpu.sync_copy(dst_hbm.at[pl.ds(ebase, EPT)], dbuf)

    def _fire(i, g80, d80, tr, ar, semt, sema):
        for q in range(K // L):
            g80[pl.ds(q * L, L)] = gbuf[pl.ds(i * K + q * L, L)]
            d80[pl.ds(q * L, L)] = dbuf[pl.ds(i * K + q * L, L)]
        pltpu.async_copy(t_hbm.at[g80], tr, semt)
        pltpu.async_copy(a_hbm.at[d80], ar, sema)

    def _process(i, g80, d80, tr, ar, semt, sema):
        pltpu.make_async_copy(t_hbm.at[g80], tr, semt).wait()
        pltpu.make_async_copy(a_hbm.at[d80], ar, sema).wait()

        def _dot(g, _):
            r16 = g * L + lax.iota(_i32, L)
            acc = jnp.zeros((L,), _f32)
            for d in range(C):
                cd = jnp.full((L,), d, _i32)
                gt = plsc.load_gather(tr, [r16, cd])
                ga = plsc.load_gather(ar, [r16, cd])
                acc = acc + gt * ga
            ob[pl.ds(i * K + g * L, L)] = acc
            return _
        lax.fori_loop(0, K // L, _dot, None)

    _fire(0, g80a, d80a, tra, ara, semta, semaa)
    _fire(1, g80b, d80b, trb, arb, semtb, semab)

    def _pair(pp, _):
        ia = 2 * pp
        _process(ia, g80a, d80a, tra, ara, semta, semaa)

        @pl.when(ia + 2 < NCH)
        def _():
            _fire(ia + 2, g80a, d80a, tra, ara, semta, semaa)

        @pl.when(ia + 1 < NCH)
        def _():
            _process(ia + 1, g80b, d80b, trb, arb, semtb, semab)

            @pl.when(ia + 3 < NCH)
            def _():
                _fire(ia + 3, g80b, d80b, trb, arb, semtb, semab)
        return _
    lax.fori_loop(0, (NCH + 1) // 2, _pair, None)

    pltpu.sync_copy(ob, out_hbm.at[pl.ds(ebase, EPT)])


def kernel(x, edge_index, edge_type, W_rel1, W_self1, b1, W_rel2, W_self2,
           b2, inter_logits, gate_log_alpha, absent_bias):
    src = edge_index[0].astype(_i32)
    dst = edge_index[1].astype(_i32)
    et = edge_type.astype(_i32)

    Y1, S1 = _enc1(x, W_rel1, W_self1, b1.reshape(1, DH))
    gsrc, sedge = _prep(src, dst, et)

    M1 = _layer128(Y1.reshape(R * N, DH), gsrc, dst, sedge)
    Y2, S2 = _enc2(S1, M1.reshape(2, N, DH), W_rel2, W_self2, b2.reshape(1, C))

    M2 = _layer128(Y2.reshape(R * N, DH), gsrc, dst, sedge)
    A, T = _enc3(S2, M2.reshape(2, N, DH), inter_logits, gate_log_alpha,
                 absent_bias.reshape(1, R))

    return _final(T.reshape(R * N, DH), A, gsrc, dst)


# speedup vs baseline: 18.8492x; 1.7947x over previous
import functools

import jax
import jax.numpy as jnp
from jax import lax
from jax.experimental import pallas as pl
from jax.experimental.pallas import tpu as pltpu
from jax.experimental.pallas import tpu_sc as plsc

N = 10000
E = 320000
R = 4
DIN = 128
DH = 128
C = 64
TEMP = 2.0 / 3.0
STRETCH_EPS = 0.1
CLIP = 2.0

NC = 2
NS = 16
L = 16

K = 80
ER = E // K
EPT = E // (NC * NS)
EPS_SC = E // NS
CH = 2000
DROWS = 512

_f32 = jnp.float32
_i32 = jnp.int32

_mesh = plsc.VectorSubcoreMesh(core_axis_name="c", subcore_axis_name="s")


BN = 2000


def _enc1_body(x_ref, wr_ref, ws_ref, b_ref, y_ref, s_ref):
    xb = x_ref[...]
    for r in range(R):
        y_ref[r] = jnp.dot(xb, wr_ref[r], preferred_element_type=_f32)
    s_ref[...] = jnp.dot(xb, ws_ref[...], preferred_element_type=_f32) + b_ref[...]


def _enc1(x, W_rel1, W_self1, b1):
    return pl.pallas_call(
        _enc1_body,
        grid=(N // BN,),
        in_specs=[
            pl.BlockSpec((BN, DIN), lambda i: (i, 0)),
            pl.BlockSpec((R, DIN, DH), lambda i: (0, 0, 0)),
            pl.BlockSpec((DIN, DH), lambda i: (0, 0)),
            pl.BlockSpec((1, DH), lambda i: (0, 0)),
        ],
        out_specs=[
            pl.BlockSpec((R, BN, DH), lambda i: (0, i, 0)),
            pl.BlockSpec((BN, DH), lambda i: (i, 0)),
        ],
        out_shape=[
            jax.ShapeDtypeStruct((R, N, DH), _f32),
            jax.ShapeDtypeStruct((N, DH), _f32),
        ],
    )(x, W_rel1, W_self1, b1)


def _enc2_body(s1_ref, m_ref, wr_ref, ws_ref, b_ref, y_ref, s_ref):
    h = jnp.maximum(s1_ref[...] + m_ref[0] + m_ref[1], 0.0)
    pad = jnp.zeros((BN, DH - C), _f32)
    for r in range(R):
        yr = jnp.dot(h, wr_ref[r], preferred_element_type=_f32)
        y_ref[r] = jnp.concatenate([yr, pad], axis=1)
    s_ref[...] = jnp.dot(h, ws_ref[...], preferred_element_type=_f32) + b_ref[...]


def _enc2(S1, M1, W_rel2, W_self2, b2):
    return pl.pallas_call(
        _enc2_body,
        grid=(N // BN,),
        in_specs=[
            pl.BlockSpec((BN, DH), lambda i: (i, 0)),
            pl.BlockSpec((2, BN, DH), lambda i: (0, i, 0)),
            pl.BlockSpec((R, DH, C), lambda i: (0, 0, 0)),
            pl.BlockSpec((DH, C), lambda i: (0, 0)),
            pl.BlockSpec((1, C), lambda i: (0, 0)),
        ],
        out_specs=[
            pl.BlockSpec((R, BN, DH), lambda i: (0, i, 0)),
            pl.BlockSpec((BN, C), lambda i: (i, 0)),
        ],
        out_shape=[
            jax.ShapeDtypeStruct((R, N, DH), _f32),
            jax.ShapeDtypeStruct((N, C), _f32),
        ],
    )(S1, M1, W_rel2, W_self2, b2)


def _enc3_body(s2_ref, m_ref, il_ref, ga_ref, ab_ref, a_ref, t_ref):
    logits = s2_ref[...] + m_ref[0][:, :C] + m_ref[1][:, :C]
    a = jax.nn.softmax(logits, axis=-1)
    pad = jnp.zeros((BN, DH - C), _f32)
    a_ref[...] = jnp.concatenate([a, pad], axis=1)
    pre = jnp.clip(ga_ref[...] / TEMP, -CLIP, CLIP)
    z = jax.nn.sigmoid(pre) * (1.0 + 2.0 * STRETCH_EPS) - STRETCH_EPS
    gate = jnp.clip(z, 0.0, 1.0)
    w = jax.nn.sigmoid(il_ref[...]) * gate
    abv = ab_ref[...]
    for r in range(R):
        tr = (jnp.dot(a, w[r], preferred_element_type=_f32)
              + abv[0:1, r:r + 1])
        t_ref[r] = jnp.concatenate([tr, pad], axis=1)


def _enc3(S2, M2, inter_logits, gate_log_alpha, ab):
    return pl.pallas_call(
        _enc3_body,
        grid=(N // BN,),
        in_specs=[
            pl.BlockSpec((BN, C), lambda i: (i, 0)),
            pl.BlockSpec((2, BN, DH), lambda i: (0, i, 0)),
            pl.BlockSpec((R, C, C), lambda i: (0, 0, 0)),
            pl.BlockSpec((R, C, C), lambda i: (0, 0, 0)),
            pl.BlockSpec((1, R), lambda i: (0, 0)),
        ],
        out_specs=[
            pl.BlockSpec((BN, DH), lambda i: (i, 0)),
            pl.BlockSpec((R, BN, DH), lambda i: (0, i, 0)),
        ],
        out_shape=[
            jax.ShapeDtypeStruct((N, DH), _f32),
            jax.ShapeDtypeStruct((R, N, DH), _f32),
        ],
    )(S2, M2, inter_logits, gate_log_alpha, ab)


@functools.partial(
    pl.kernel,
    out_type=(
        jax.ShapeDtypeStruct((E,), _i32),
        jax.ShapeDtypeStruct((E,), _f32),
    ),
    mesh=_mesh,
    compiler_params=pltpu.CompilerParams(needs_layout_passes=False),
    scratch_types=[
        pltpu.VMEM((DROWS, 128), _f32),
        pltpu.VMEM((CH,), _i32),
        pltpu.VMEM((CH,), _i32),
        pltpu.VMEM((CH,), _i32),
        pltpu.VMEM((32, 128), _i32),
        pltpu.VMEM((CH,), _i32),
        pltpu.VMEM((CH,), _f32),
        pltpu.VMEM_SHARED((DROWS, 128), _f32),
    ],
)
def _prep(src_hbm, dst_hbm, et_hbm, gsrc_hbm, sedge_hbm,
          deg_t, srcb, dstb, etb, idxrows, gout, sout, deg_s):
    cid = lax.axis_index("c")
    sid = lax.axis_index("s")
    zero16 = jnp.zeros((L,), _f32)

    def _z(j, _):
        for q in range(128 // L):
            deg_t[j, pl.ds(q * L, L)] = zero16
        return _
    lax.fori_loop(0, DROWS, _z, None)

    @pl.when(sid == 0)
    def _():
        pltpu.sync_copy(deg_t, deg_s)
    plsc.subcore_barrier()

    def _hist_chunk(ci, _):
        base = sid * EPS_SC + ci * CH
        pltpu.sync_copy(dst_hbm.at[pl.ds(base, CH)], dstb)
        pltpu.sync_copy(et_hbm.at[pl.ds(base, CH)], etb)

        def _grp(g, _):
            d16 = dstb[pl.ds(g * L, L)]
            e16 = etb[pl.ds(g * L, L)]
            idx = e16 * N + d16
            cnt, lastm = plsc.scan_count(idx)
            row = lax.shift_right_logical(idx, 7)
            col = jnp.bitwise_and(idx, 127)
            plsc.addupdate_scatter(deg_t, [row, col], cnt.astype(_f32),
                                   mask=lastm)
            return _
        lax.fori_loop(0, CH // L, _grp, None)
        return _
    lax.fori_loop(0, EPS_SC // CH, _hist_chunk, None)
    plsc.subcore_barrier()

    iota16 = lax.iota(_i32, L)
    for cc in range(4):
        for gg in range(8):
            idxrows[cc * 8, pl.ds(gg * L, L)] = iota16 + (cc * 128 + gg * L)
    for cc in range(4):
        pltpu.sync_copy(deg_t.at[pl.ds(cc * 128, 128)],
                        deg_s.at[idxrows.at[cc * 8]], add=True)
    plsc.subcore_barrier()

    pltpu.sync_copy(deg_s, deg_t)

    def _s(j, _):
        for q in range(128 // L):
            v = deg_t[j, pl.ds(q * L, L)]
            deg_t[j, pl.ds(q * L, L)] = 1.0 / jnp.maximum(v, 1.0)
        return _
    lax.fori_loop(0, DROWS, _s, None)

    gwid = cid * NS + sid

    def _out_chunk(ci, _):
        base = gwid * EPT + ci * CH
        pltpu.sync_copy(src_hbm.at[pl.ds(base, CH)], srcb)
        pltpu.sync_copy(dst_hbm.at[pl.ds(base, CH)], dstb)
        pltpu.sync_copy(et_hbm.at[pl.ds(base, CH)], etb)

        def _grp(g, _):
            s16 = srcb[pl.ds(g * L, L)]
            d16 = dstb[pl.ds(g * L, L)]
            e16 = etb[pl.ds(g * L, L)]
            gout[pl.ds(g * L, L)] = e16 * N + s16
            sidx = e16 * N + d16
            srow = lax.shift_right_logical(sidx, 7)
            scol = jnp.bitwise_and(sidx, 127)
            sout[pl.ds(g * L, L)] = plsc.load_gather(deg_t, [srow, scol])
            return _
        lax.fori_loop(0, CH // L, _grp, None)
        pltpu.sync_copy(gout, gsrc_hbm.at[pl.ds(base, CH)])
        pltpu.sync_copy(sout, sedge_hbm.at[pl.ds(base, CH)])
        return _
    lax.fori_loop(0, EPT // CH, _out_chunk, None)


def _make_layer(D):
    RCH = N // K
    NCH = CH // K
    NB = 3

    @functools.partial(
        pl.kernel,
        out_type=jax.ShapeDtypeStruct((2 * N, D), _f32),
        mesh=_mesh,
        compiler_params=pltpu.CompilerParams(needs_layout_passes=False),
        scratch_types=[
            pltpu.VMEM((CH,), _i32),
            pltpu.VMEM((CH,), _i32),
            pltpu.VMEM((CH,), _f32),
            [pltpu.VMEM((K,), _i32)] * NB,
            [pltpu.VMEM((K,), _i32)] * NB,
            [pltpu.VMEM((K, D), _f32)] * NB,
            [pltpu.SemaphoreType.DMA] * NB,
            [pltpu.SemaphoreType.DMA] * NB,
            pltpu.VMEM_SHARED((N, D), _f32),
        ],
    )
    def _layer(y_hbm, gsrc_hbm, dst_hbm, sed_hbm, m_hbm,
               gbuf, dbuf, sbuf, g80s, d80s, rowss, gsems, ssems, msh):
        cid = lax.axis_index("c")
        sid = lax.axis_index("s")
        zero16 = jnp.zeros((L,), _f32)

        rows0 = rowss[0]
        for j in range(K):
            for d in range(D // L):
                rows0[j, pl.ds(d * L, L)] = zero16
        nu = (RCH - sid + NS - 1) // NS

        def _z(k, _):
            u = sid + k * NS
            pltpu.sync_copy(rows0, msh.at[pl.ds(u * K, K)])
            return _
        lax.fori_loop(0, nu, _z, None)
        plsc.subcore_barrier()

        ebase = cid * (E // 2) + sid * EPT

        def _fire(i, sl):
            g80, d80 = g80s[sl], d80s[sl]
            for q in range(K // L):
                g80[pl.ds(q * L, L)] = gbuf[pl.ds(i * K + q * L, L)]
                d80[pl.ds(q * L, L)] = dbuf[pl.ds(i * K + q * L, L)]
            pltpu.async_copy(y_hbm.at[g80], rowss[sl], gsems[sl])

        def _process(i, sl):
            rows = rowss[sl]
            pltpu.make_async_copy(y_hbm.at[g80s[sl]], rows, gsems[sl]).wait()

            def _scale(g, _):
                s16 = sbuf[pl.ds(i * K + g * L, L)]
                for jj in range(L):
                    sv = jnp.full((L,), s16[jj], _f32)
                    j = g * L + jj
                    for d in range(D // L):
                        rows[j, pl.ds(d * L, L)] = rows[j, pl.ds(d * L, L)] * sv
                return _
            lax.fori_loop(0, K // L, _scale, None)
            pltpu.async_copy(rows, msh.at[d80s[sl]], ssems[sl], add=True)

        def _wait_scat(sl):
            pltpu.make_async_copy(rowss[sl], msh.at[d80s[sl]],
                                  ssems[sl]).wait()

        def _blk(t, _):
            base = ebase + t * CH
            pltpu.sync_copy(gsrc_hbm.at[pl.ds(base, CH)], gbuf)
            pltpu.sync_copy(dst_hbm.at[pl.ds(base, CH)], dbuf)
            pltpu.sync_copy(sed_hbm.at[pl.ds(base, CH)], sbuf)
            for sl in range(NB):
                _fire(sl, sl)

            def _trip(pp, _):
                i0 = NB * pp
                for off in range(NB):
                    i = i0 + off

                    @pl.when(i < NCH)
                    def _():
                        _process(i, off)
                for off in range(NB):
                    j = i0 + NB + off

                    @pl.when(j < NCH)
                    def _():
                        _wait_scat(off)
                        _fire(j, off)
                return _
            lax.fori_loop(0, (NCH + NB - 1) // NB, _trip, None)
            for sl in range(NB):
                _wait_scat(sl)
            return _
        lax.fori_loop(0, EPT // CH, _blk, None)
        plsc.subcore_barrier()

        def _wb(k, _):
            off = (sid + k * NS) * K
            pltpu.sync_copy(msh.at[pl.ds(off, K)],
                            m_hbm.at[pl.ds(cid * N + off, K)])
            return _
        lax.fori_loop(0, nu, _wb, None)

    return _layer


_layer128 = _make_layer(DH)


@functools.partial(
    pl.kernel,
    out_type=jax.ShapeDtypeStruct((E,), _f32),
    mesh=_mesh,
    compiler_params=pltpu.CompilerParams(needs_layout_passes=False),
    scratch_types=[
        pltpu.VMEM((EPT,), _i32),
        pltpu.VMEM((EPT,), _i32),
        pltpu.VMEM((K,), _i32),
        pltpu.VMEM((K,), _i32),
        pltpu.VMEM((K,), _i32),
        pltpu.VMEM((K,), _i32),
        pltpu.VMEM((K, DH), _f32),
        pltpu.VMEM((K, DH), _f32),
        pltpu.VMEM((K, DH), _f32),
        pltpu.VMEM((K, DH), _f32),
        pltpu.VMEM((EPT,), _f32),
        pltpu.SemaphoreType.DMA,
        pltpu.SemaphoreType.DMA,
        pltpu.SemaphoreType.DMA,
        pltpu.SemaphoreType.DMA,
    ],
)
def _final(t_hbm, a_hbm, gsrc_hbm, dst_hbm, out_hbm,
           gbuf, dbuf, g80a, g80b, d80a, d80b, tra, trb, ara, arb, ob,
           semta, semtb, semaa, semab):
    cid = lax.axis_index("c")
    sid = lax.axis_index("s")
    gwid = cid * NS + sid
    ebase = gwid * EPT
    NCH = EPT // K

    pltpu.sync_copy(gsrc_hbm.at[pl.ds(ebase, EPT)], gbuf)
    pltpu.sync_copy(dst_hbm.at[pl.ds(ebase, EPT)], dbuf)

    def _fire(i, g80, d80, tr, ar, semt, sema):
        for q in range(K // L):
            g80[pl.ds(q * L, L)] = gbuf[pl.ds(i * K + q * L, L)]
            d80[pl.ds(q * L, L)] = dbuf[pl.ds(i * K + q * L, L)]
        pltpu.async_copy(t_hbm.at[g80], tr, semt)
        pltpu.async_copy(a_hbm.at[d80], ar, sema)

    def _process(i, g80, d80, tr, ar, semt, sema):
        pltpu.make_async_copy(t_hbm.at[g80], tr, semt).wait()
        pltpu.make_async_copy(a_hbm.at[d80], ar, sema).wait()

        lanes = lax.iota(_i32, L)

        def _dot(g, _):
            outv = jnp.zeros((L,), _f32)
            for jj in range(L):
                j = g * L + jj
                pr = tr[j, pl.ds(0, L)] * ar[j, pl.ds(0, L)]
                for d in range(1, C // L):
                    pr = pr + tr[j, pl.ds(d * L, L)] * ar[j, pl.ds(d * L, L)]
                outv = jnp.where(lanes == jj, jnp.sum(pr), outv)
            ob[pl.ds(i * K + g * L, L)] = outv
            return _
        lax.fori_loop(0, K // L, _dot, None)

    _fire(0, g80a, d80a, tra, ara, semta, semaa)
    _fire(1, g80b, d80b, trb, arb, semtb, semab)

    def _pair(pp, _):
        ia = 2 * pp
        _process(ia, g80a, d80a, tra, ara, semta, semaa)

        @pl.when(ia + 2 < NCH)
        def _():
            _fire(ia + 2, g80a, d80a, tra, ara, semta, semaa)

        @pl.when(ia + 1 < NCH)
        def _():
            _process(ia + 1, g80b, d80b, trb, arb, semtb, semab)

            @pl.when(ia + 3 < NCH)
            def _():
                _fire(ia + 3, g80b, d80b, trb, arb, semtb, semab)
        return _
    lax.fori_loop(0, (NCH + 1) // 2, _pair, None)

    pltpu.sync_copy(ob, out_hbm.at[pl.ds(ebase, EPT)])


def kernel(x, edge_index, edge_type, W_rel1, W_self1, b1, W_rel2, W_self2,
           b2, inter_logits, gate_log_alpha, absent_bias):
    src = edge_index[0].astype(_i32)
    dst = edge_index[1].astype(_i32)
    et = edge_type.astype(_i32)

    Y1, S1 = _enc1(x, W_rel1, W_self1, b1.reshape(1, DH))
    gsrc, sedge = _prep(src, dst, et)

    M1 = _layer128(Y1.reshape(R * N, DH), gsrc, dst, sedge)
    Y2, S2 = _enc2(S1, M1.reshape(2, N, DH), W_rel2, W_self2, b2.reshape(1, C))

    M2 = _layer128(Y2.reshape(R * N, DH), gsrc, dst, sedge)
    A, T = _enc3(S2, M2.reshape(2, N, DH), inter_logits, gate_log_alpha,
                 absent_bias.reshape(1, R))

    return _final(T.reshape(R * N, DH), A, gsrc, dst)


# speedup vs baseline: 19.7700x; 1.0489x over previous
import functools

import jax
import jax.numpy as jnp
from jax import lax
from jax.experimental import pallas as pl
from jax.experimental.pallas import tpu as pltpu
from jax.experimental.pallas import tpu_sc as plsc

N = 10000
E = 320000
R = 4
DIN = 128
DH = 128
C = 64
TEMP = 2.0 / 3.0
STRETCH_EPS = 0.1
CLIP = 2.0

NC = 2
NS = 16
L = 16

K = 80
ER = E // K
EPT = E // (NC * NS)
EPS_SC = E // NS
CH = 2000
DROWS = 512

_f32 = jnp.float32
_i32 = jnp.int32

_mesh = plsc.VectorSubcoreMesh(core_axis_name="c", subcore_axis_name="s")


BN = 2000


def _enc1_body(x_ref, wr_ref, ws_ref, b_ref, y_ref, s_ref):
    xb = x_ref[...]
    for r in range(R):
        y_ref[r] = jnp.dot(xb, wr_ref[r], preferred_element_type=_f32)
    s_ref[...] = jnp.dot(xb, ws_ref[...], preferred_element_type=_f32) + b_ref[...]


def _enc1(x, W_rel1, W_self1, b1):
    return pl.pallas_call(
        _enc1_body,
        grid=(N // BN,),
        in_specs=[
            pl.BlockSpec((BN, DIN), lambda i: (i, 0)),
            pl.BlockSpec((R, DIN, DH), lambda i: (0, 0, 0)),
            pl.BlockSpec((DIN, DH), lambda i: (0, 0)),
            pl.BlockSpec((1, DH), lambda i: (0, 0)),
        ],
        out_specs=[
            pl.BlockSpec((R, BN, DH), lambda i: (0, i, 0)),
            pl.BlockSpec((BN, DH), lambda i: (i, 0)),
        ],
        out_shape=[
            jax.ShapeDtypeStruct((R, N, DH), _f32),
            jax.ShapeDtypeStruct((N, DH), _f32),
        ],
    )(x, W_rel1, W_self1, b1)


def _enc2_body(s1_ref, m_ref, wr_ref, ws_ref, b_ref, y_ref, s_ref):
    h = jnp.maximum(s1_ref[...] + m_ref[0] + m_ref[1], 0.0)
    pad = jnp.zeros((BN, DH - C), _f32)
    for r in range(R):
        yr = jnp.dot(h, wr_ref[r], preferred_element_type=_f32)
        y_ref[r] = jnp.concatenate([yr, pad], axis=1)
    s_ref[...] = jnp.dot(h, ws_ref[...], preferred_element_type=_f32) + b_ref[...]


def _enc2(S1, M1, W_rel2, W_self2, b2):
    return pl.pallas_call(
        _enc2_body,
        grid=(N // BN,),
        in_specs=[
            pl.BlockSpec((BN, DH), lambda i: (i, 0)),
            pl.BlockSpec((2, BN, DH), lambda i: (0, i, 0)),
            pl.BlockSpec((R, DH, C), lambda i: (0, 0, 0)),
            pl.BlockSpec((DH, C), lambda i: (0, 0)),
            pl.BlockSpec((1, C), lambda i: (0, 0)),
        ],
        out_specs=[
            pl.BlockSpec((R, BN, DH), lambda i: (0, i, 0)),
            pl.BlockSpec((BN, C), lambda i: (i, 0)),
        ],
        out_shape=[
            jax.ShapeDtypeStruct((R, N, DH), _f32),
            jax.ShapeDtypeStruct((N, C), _f32),
        ],
    )(S1, M1, W_rel2, W_self2, b2)


def _enc3_body(s2_ref, m_ref, il_ref, ga_ref, ab_ref, a_ref, t_ref):
    logits = s2_ref[...] + m_ref[0][:, :C] + m_ref[1][:, :C]
    a = jax.nn.softmax(logits, axis=-1)
    pad = jnp.zeros((BN, DH - C), _f32)
    a_ref[...] = jnp.concatenate([a, pad], axis=1)
    pre = jnp.clip(ga_ref[...] / TEMP, -CLIP, CLIP)
    z = jax.nn.sigmoid(pre) * (1.0 + 2.0 * STRETCH_EPS) - STRETCH_EPS
    gate = jnp.clip(z, 0.0, 1.0)
    w = jax.nn.sigmoid(il_ref[...]) * gate
    abv = ab_ref[...]
    for r in range(R):
        tr = (jnp.dot(a, w[r], preferred_element_type=_f32)
              + abv[0:1, r:r + 1])
        t_ref[r] = jnp.concatenate([tr, pad], axis=1)


def _enc3(S2, M2, inter_logits, gate_log_alpha, ab):
    return pl.pallas_call(
        _enc3_body,
        grid=(N // BN,),
        in_specs=[
            pl.BlockSpec((BN, C), lambda i: (i, 0)),
            pl.BlockSpec((2, BN, DH), lambda i: (0, i, 0)),
            pl.BlockSpec((R, C, C), lambda i: (0, 0, 0)),
            pl.BlockSpec((R, C, C), lambda i: (0, 0, 0)),
            pl.BlockSpec((1, R), lambda i: (0, 0)),
        ],
        out_specs=[
            pl.BlockSpec((BN, DH), lambda i: (i, 0)),
            pl.BlockSpec((R, BN, DH), lambda i: (0, i, 0)),
        ],
        out_shape=[
            jax.ShapeDtypeStruct((N, DH), _f32),
            jax.ShapeDtypeStruct((R, N, DH), _f32),
        ],
    )(S2, M2, inter_logits, gate_log_alpha, ab)


@functools.partial(
    pl.kernel,
    out_type=(
        jax.ShapeDtypeStruct((E,), _i32),
        jax.ShapeDtypeStruct((E,), _f32),
    ),
    mesh=_mesh,
    compiler_params=pltpu.CompilerParams(needs_layout_passes=False),
    scratch_types=[
        pltpu.VMEM((DROWS, 128), _f32),
        pltpu.VMEM((CH,), _i32),
        pltpu.VMEM((CH,), _i32),
        pltpu.VMEM((CH,), _i32),
        pltpu.VMEM((32, 128), _i32),
        pltpu.VMEM((CH,), _i32),
        pltpu.VMEM((CH,), _f32),
        pltpu.VMEM_SHARED((DROWS, 128), _f32),
    ],
)
def _prep(src_hbm, dst_hbm, et_hbm, gsrc_hbm, sedge_hbm,
          deg_t, srcb, dstb, etb, idxrows, gout, sout, deg_s):
    cid = lax.axis_index("c")
    sid = lax.axis_index("s")
    zero16 = jnp.zeros((L,), _f32)

    def _z(j, _):
        for q in range(128 // L):
            deg_t[j, pl.ds(q * L, L)] = zero16
        return _
    lax.fori_loop(0, DROWS, _z, None)

    @pl.when(sid == 0)
    def _():
        pltpu.sync_copy(deg_t, deg_s)
    plsc.subcore_barrier()

    def _hist_chunk(ci, _):
        base = sid * EPS_SC + ci * CH
        pltpu.sync_copy(dst_hbm.at[pl.ds(base, CH)], dstb)
        pltpu.sync_copy(et_hbm.at[pl.ds(base, CH)], etb)

        def _grp(g, _):
            d16 = dstb[pl.ds(g * L, L)]
            e16 = etb[pl.ds(g * L, L)]
            idx = e16 * N + d16
            cnt, lastm = plsc.scan_count(idx)
            row = lax.shift_right_logical(idx, 7)
            col = jnp.bitwise_and(idx, 127)
            plsc.addupdate_scatter(deg_t, [row, col], cnt.astype(_f32),
                                   mask=lastm)
            return _
        lax.fori_loop(0, CH // L, _grp, None)
        return _
    lax.fori_loop(0, EPS_SC // CH, _hist_chunk, None)
    plsc.subcore_barrier()

    iota16 = lax.iota(_i32, L)
    for cc in range(4):
        for gg in range(8):
            idxrows[cc * 8, pl.ds(gg * L, L)] = iota16 + (cc * 128 + gg * L)
    for cc in range(4):
        pltpu.sync_copy(deg_t.at[pl.ds(cc * 128, 128)],
                        deg_s.at[idxrows.at[cc * 8]], add=True)
    plsc.subcore_barrier()

    pltpu.sync_copy(deg_s, deg_t)

    def _s(j, _):
        for q in range(128 // L):
            v = deg_t[j, pl.ds(q * L, L)]
            deg_t[j, pl.ds(q * L, L)] = 1.0 / jnp.maximum(v, 1.0)
        return _
    lax.fori_loop(0, DROWS, _s, None)

    gwid = cid * NS + sid

    def _out_chunk(ci, _):
        base = gwid * EPT + ci * CH
        pltpu.sync_copy(src_hbm.at[pl.ds(base, CH)], srcb)
        pltpu.sync_copy(dst_hbm.at[pl.ds(base, CH)], dstb)
        pltpu.sync_copy(et_hbm.at[pl.ds(base, CH)], etb)

        def _grp(g, _):
            s16 = srcb[pl.ds(g * L, L)]
            d16 = dstb[pl.ds(g * L, L)]
            e16 = etb[pl.ds(g * L, L)]
            gout[pl.ds(g * L, L)] = e16 * N + s16
            sidx = e16 * N + d16
            srow = lax.shift_right_logical(sidx, 7)
            scol = jnp.bitwise_and(sidx, 127)
            sout[pl.ds(g * L, L)] = plsc.load_gather(deg_t, [srow, scol])
            return _
        lax.fori_loop(0, CH // L, _grp, None)
        pltpu.sync_copy(gout, gsrc_hbm.at[pl.ds(base, CH)])
        pltpu.sync_copy(sout, sedge_hbm.at[pl.ds(base, CH)])
        return _
    lax.fori_loop(0, EPT // CH, _out_chunk, None)


def _make_layer(D):
    RCH = N // K
    NCH = CH // K
    NB = 3

    @functools.partial(
        pl.kernel,
        out_type=jax.ShapeDtypeStruct((2 * N, D), _f32),
        mesh=_mesh,
        compiler_params=pltpu.CompilerParams(needs_layout_passes=False),
        scratch_types=[
            pltpu.VMEM((CH,), _i32),
            pltpu.VMEM((CH,), _i32),
            pltpu.VMEM((CH,), _f32),
            [pltpu.VMEM((K,), _i32)] * NB,
            [pltpu.VMEM((K,), _i32)] * NB,
            [pltpu.VMEM((K, D), _f32)] * NB,
            [pltpu.SemaphoreType.DMA] * NB,
            [pltpu.SemaphoreType.DMA] * NB,
            pltpu.VMEM_SHARED((N, D), _f32),
        ],
    )
    def _layer(y_hbm, gsrc_hbm, dst_hbm, sed_hbm, m_hbm,
               gbuf, dbuf, sbuf, g80s, d80s, rowss, gsems, ssems, msh):
        cid = lax.axis_index("c")
        sid = lax.axis_index("s")
        zero16 = jnp.zeros((L,), _f32)

        rows0 = rowss[0]
        for j in range(K):
            for d in range(D // L):
                rows0[j, pl.ds(d * L, L)] = zero16
        nu = (RCH - sid + NS - 1) // NS

        def _z(k, _):
            u = sid + k * NS
            pltpu.sync_copy(rows0, msh.at[pl.ds(u * K, K)])
            return _
        lax.fori_loop(0, nu, _z, None)
        plsc.subcore_barrier()

        ebase = cid * (E // 2) + sid * EPT

        def _fire(i, sl):
            g80, d80 = g80s[sl], d80s[sl]
            for q in range(K // L):
                g80[pl.ds(q * L, L)] = gbuf[pl.ds(i * K + q * L, L)]
                d80[pl.ds(q * L, L)] = dbuf[pl.ds(i * K + q * L, L)]
            pltpu.async_copy(y_hbm.at[g80], rowss[sl], gsems[sl])

        def _process(i, sl):
            rows = rowss[sl]
            pltpu.make_async_copy(y_hbm.at[g80s[sl]], rows, gsems[sl]).wait()

            def _scale(g, _):
                s16 = sbuf[pl.ds(i * K + g * L, L)]
                for jj in range(L):
                    sv = jnp.full((L,), s16[jj], _f32)
                    j = g * L + jj
                    for d in range(D // L):
                        rows[j, pl.ds(d * L, L)] = rows[j, pl.ds(d * L, L)] * sv
                return _
            lax.fori_loop(0, K // L, _scale, None)
            pltpu.async_copy(rows, msh.at[d80s[sl]], ssems[sl], add=True)

        def _wait_scat(sl):
            pltpu.make_async_copy(rowss[sl], msh.at[d80s[sl]],
                                  ssems[sl]).wait()

        def _blk(t, _):
            base = ebase + t * CH
            pltpu.sync_copy(gsrc_hbm.at[pl.ds(base, CH)], gbuf)
            pltpu.sync_copy(dst_hbm.at[pl.ds(base, CH)], dbuf)
            pltpu.sync_copy(sed_hbm.at[pl.ds(base, CH)], sbuf)
            for sl in range(NB):
                _fire(sl, sl)

            def _trip(pp, _):
                i0 = NB * pp
                for off in range(NB):
                    i = i0 + off

                    @pl.when(i < NCH)
                    def _():
                        _process(i, off)
                for off in range(NB):
                    j = i0 + NB + off

                    @pl.when(j < NCH)
                    def _():
                        _wait_scat(off)
                        _fire(j, off)
                return _
            lax.fori_loop(0, (NCH + NB - 1) // NB, _trip, None)
            for sl in range(NB):
                _wait_scat(sl)
            return _
        lax.fori_loop(0, EPT // CH, _blk, None)
        plsc.subcore_barrier()

        def _wb(k, _):
            off = (sid + k * NS) * K
            pltpu.sync_copy(msh.at[pl.ds(off, K)],
                            m_hbm.at[pl.ds(cid * N + off, K)])
            return _
        lax.fori_loop(0, nu, _wb, None)

    return _layer


_layer128 = _make_layer(DH)


@functools.partial(
    pl.kernel,
    out_type=jax.ShapeDtypeStruct((E,), _f32),
    mesh=_mesh,
    compiler_params=pltpu.CompilerParams(needs_layout_passes=False),
    scratch_types=[
        pltpu.VMEM((CH,), _i32),
        pltpu.VMEM((CH,), _i32),
        pltpu.VMEM((K,), _i32),
        pltpu.VMEM((K,), _i32),
        pltpu.VMEM((K,), _i32),
        pltpu.VMEM((K,), _i32),
        pltpu.VMEM((K, DH), _f32),
        pltpu.VMEM((K, DH), _f32),
        pltpu.VMEM((K, DH), _f32),
        pltpu.VMEM((K, DH), _f32),
        pltpu.VMEM((CH,), _f32),
        pltpu.SemaphoreType.DMA,
        pltpu.SemaphoreType.DMA,
        pltpu.SemaphoreType.DMA,
        pltpu.SemaphoreType.DMA,
        pltpu.VMEM_SHARED((N, DH), _f32),
    ],
)
def _final(t_hbm, a_hbm, gsrc_hbm, dst_hbm, out_hbm,
           gbuf, dbuf, g80a, g80b, d80a, d80b, tra, trb, ara, arb, ob,
           semta, semtb, semaa, semab, ashr):
    cid = lax.axis_index("c")
    sid = lax.axis_index("s")
    gwid = cid * NS + sid
    ebase = gwid * EPT
    NCH = CH // K

    nu = ((N // K) - sid + NS - 1) // NS

    def _st(k, _):
        off = (sid + k * NS) * K
        pltpu.sync_copy(a_hbm.at[pl.ds(off, K)], ashr.at[pl.ds(off, K)])
        return _
    lax.fori_loop(0, nu, _st, None)
    plsc.subcore_barrier()

    def _fire(i, g80, d80, tr, ar, semt, sema):
        for q in range(K // L):
            g80[pl.ds(q * L, L)] = gbuf[pl.ds(i * K + q * L, L)]
            d80[pl.ds(q * L, L)] = dbuf[pl.ds(i * K + q * L, L)]
        pltpu.async_copy(t_hbm.at[g80], tr, semt)
        pltpu.async_copy(ashr.at[d80], ar, sema)

    def _process(i, g80, d80, tr, ar, semt, sema):
        pltpu.make_async_copy(t_hbm.at[g80], tr, semt).wait()
        pltpu.make_async_copy(ashr.at[d80], ar, sema).wait()
        lanes = lax.iota(_i32, L)

        def _dot(g, _):
            outv = jnp.zeros((L,), _f32)
            for jj in range(L):
                j = g * L + jj
                pr = tr[j, pl.ds(0, L)] * ar[j, pl.ds(0, L)]
                for d in range(1, C // L):
                    pr = pr + tr[j, pl.ds(d * L, L)] * ar[j, pl.ds(d * L, L)]
                outv = jnp.where(lanes == jj, jnp.sum(pr), outv)
            ob[pl.ds(i * K + g * L, L)] = outv
            return _
        lax.fori_loop(0, K // L, _dot, None)

    def _blk(t, _):
        base = ebase + t * CH
        pltpu.sync_copy(gsrc_hbm.at[pl.ds(base, CH)], gbuf)
        pltpu.sync_copy(dst_hbm.at[pl.ds(base, CH)], dbuf)
        _fire(0, g80a, d80a, tra, ara, semta, semaa)
        _fire(1, g80b, d80b, trb, arb, semtb, semab)

        def _pair(pp, _):
            ia = 2 * pp
            _process(ia, g80a, d80a, tra, ara, semta, semaa)

            @pl.when(ia + 2 < NCH)
            def _():
                _fire(ia + 2, g80a, d80a, tra, ara, semta, semaa)

            @pl.when(ia + 1 < NCH)
            def _():
                _process(ia + 1, g80b, d80b, trb, arb, semtb, semab)

                @pl.when(ia + 3 < NCH)
                def _():
                    _fire(ia + 3, g80b, d80b, trb, arb, semtb, semab)
            return _
        lax.fori_loop(0, (NCH + 1) // 2, _pair, None)
        pltpu.sync_copy(ob, out_hbm.at[pl.ds(base, CH)])
        return _
    lax.fori_loop(0, EPT // CH, _blk, None)


def kernel(x, edge_index, edge_type, W_rel1, W_self1, b1, W_rel2, W_self2,
           b2, inter_logits, gate_log_alpha, absent_bias):
    src = edge_index[0].astype(_i32)
    dst = edge_index[1].astype(_i32)
    et = edge_type.astype(_i32)

    Y1, S1 = _enc1(x, W_rel1, W_self1, b1.reshape(1, DH))
    gsrc, sedge = _prep(src, dst, et)

    M1 = _layer128(Y1.reshape(R * N, DH), gsrc, dst, sedge)
    Y2, S2 = _enc2(S1, M1.reshape(2, N, DH), W_rel2, W_self2, b2.reshape(1, C))

    M2 = _layer128(Y2.reshape(R * N, DH), gsrc, dst, sedge)
    A, T = _enc3(S2, M2.reshape(2, N, DH), inter_logits, gate_log_alpha,
                 absent_bias.reshape(1, R))

    return _final(T.reshape(R * N, DH), A, gsrc, dst)


# speedup vs baseline: 19.8314x; 1.0031x over previous
import functools

import jax
import jax.numpy as jnp
from jax import lax
from jax.experimental import pallas as pl
from jax.experimental.pallas import tpu as pltpu
from jax.experimental.pallas import tpu_sc as plsc

N = 10000
E = 320000
R = 4
DIN = 128
DH = 128
C = 64
TEMP = 2.0 / 3.0
STRETCH_EPS = 0.1
CLIP = 2.0

NC = 2
NS = 16
L = 16

K = 80
ER = E // K
EPT = E // (NC * NS)
EPS_SC = E // NS
CH = 2000
DROWS = 512

_f32 = jnp.float32
_i32 = jnp.int32

_mesh = plsc.VectorSubcoreMesh(core_axis_name="c", subcore_axis_name="s")


BN = 2000


def _enc1_body(x_ref, wr_ref, ws_ref, b_ref, y_ref, s_ref):
    xb = x_ref[...]
    for r in range(R):
        y_ref[r] = jnp.dot(xb, wr_ref[r], preferred_element_type=_f32)
    s_ref[...] = jnp.dot(xb, ws_ref[...], preferred_element_type=_f32) + b_ref[...]


def _enc1(x, W_rel1, W_self1, b1):
    return pl.pallas_call(
        _enc1_body,
        grid=(N // BN,),
        in_specs=[
            pl.BlockSpec((BN, DIN), lambda i: (i, 0)),
            pl.BlockSpec((R, DIN, DH), lambda i: (0, 0, 0)),
            pl.BlockSpec((DIN, DH), lambda i: (0, 0)),
            pl.BlockSpec((1, DH), lambda i: (0, 0)),
        ],
        out_specs=[
            pl.BlockSpec((R, BN, DH), lambda i: (0, i, 0)),
            pl.BlockSpec((BN, DH), lambda i: (i, 0)),
        ],
        out_shape=[
            jax.ShapeDtypeStruct((R, N, DH), _f32),
            jax.ShapeDtypeStruct((N, DH), _f32),
        ],
    )(x, W_rel1, W_self1, b1)


def _enc2_body(s1_ref, m_ref, wr_ref, ws_ref, b_ref, y_ref, s_ref):
    h = jnp.maximum(s1_ref[...] + m_ref[0] + m_ref[1], 0.0)
    pad = jnp.zeros((BN, DH - C), _f32)
    for r in range(R):
        yr = jnp.dot(h, wr_ref[r], preferred_element_type=_f32)
        y_ref[r] = jnp.concatenate([yr, pad], axis=1)
    s_ref[...] = jnp.dot(h, ws_ref[...], preferred_element_type=_f32) + b_ref[...]


def _enc2(S1, M1, W_rel2, W_self2, b2):
    return pl.pallas_call(
        _enc2_body,
        grid=(N // BN,),
        in_specs=[
            pl.BlockSpec((BN, DH), lambda i: (i, 0)),
            pl.BlockSpec((2, BN, DH), lambda i: (0, i, 0)),
            pl.BlockSpec((R, DH, C), lambda i: (0, 0, 0)),
            pl.BlockSpec((DH, C), lambda i: (0, 0)),
            pl.BlockSpec((1, C), lambda i: (0, 0)),
        ],
        out_specs=[
            pl.BlockSpec((R, BN, DH), lambda i: (0, i, 0)),
            pl.BlockSpec((BN, C), lambda i: (i, 0)),
        ],
        out_shape=[
            jax.ShapeDtypeStruct((R, N, DH), _f32),
            jax.ShapeDtypeStruct((N, C), _f32),
        ],
    )(S1, M1, W_rel2, W_self2, b2)


def _enc3_body(s2_ref, m_ref, il_ref, ga_ref, ab_ref, a_ref, t_ref):
    logits = s2_ref[...] + m_ref[0][:, :C] + m_ref[1][:, :C]
    a = jax.nn.softmax(logits, axis=-1)
    pad = jnp.zeros((BN, DH - C), _f32)
    a_ref[...] = jnp.concatenate([a, pad], axis=1)
    pre = jnp.clip(ga_ref[...] / TEMP, -CLIP, CLIP)
    z = jax.nn.sigmoid(pre) * (1.0 + 2.0 * STRETCH_EPS) - STRETCH_EPS
    gate = jnp.clip(z, 0.0, 1.0)
    w = jax.nn.sigmoid(il_ref[...]) * gate
    abv = ab_ref[...]
    for r in range(R):
        tr = (jnp.dot(a, w[r], preferred_element_type=_f32)
              + abv[0:1, r:r + 1])
        t_ref[r] = jnp.concatenate([tr, pad], axis=1)


def _enc3(S2, M2, inter_logits, gate_log_alpha, ab):
    return pl.pallas_call(
        _enc3_body,
        grid=(N // BN,),
        in_specs=[
            pl.BlockSpec((BN, C), lambda i: (i, 0)),
            pl.BlockSpec((2, BN, DH), lambda i: (0, i, 0)),
            pl.BlockSpec((R, C, C), lambda i: (0, 0, 0)),
            pl.BlockSpec((R, C, C), lambda i: (0, 0, 0)),
            pl.BlockSpec((1, R), lambda i: (0, 0)),
        ],
        out_specs=[
            pl.BlockSpec((BN, DH), lambda i: (i, 0)),
            pl.BlockSpec((R, BN, DH), lambda i: (0, i, 0)),
        ],
        out_shape=[
            jax.ShapeDtypeStruct((N, DH), _f32),
            jax.ShapeDtypeStruct((R, N, DH), _f32),
        ],
    )(S2, M2, inter_logits, gate_log_alpha, ab)


@functools.partial(
    pl.kernel,
    out_type=(
        jax.ShapeDtypeStruct((E,), _i32),
        jax.ShapeDtypeStruct((E,), _f32),
    ),
    mesh=_mesh,
    compiler_params=pltpu.CompilerParams(needs_layout_passes=False),
    scratch_types=[
        pltpu.VMEM((DROWS, 128), _f32),
        pltpu.VMEM((CH,), _i32),
        pltpu.VMEM((CH,), _i32),
        pltpu.VMEM((CH,), _i32),
        pltpu.VMEM((32, 128), _i32),
        pltpu.VMEM((CH,), _i32),
        pltpu.VMEM((CH,), _f32),
        pltpu.VMEM_SHARED((DROWS, 128), _f32),
    ],
)
def _prep(src_hbm, dst_hbm, et_hbm, gsrc_hbm, sedge_hbm,
          deg_t, srcb, dstb, etb, idxrows, gout, sout, deg_s):
    cid = lax.axis_index("c")
    sid = lax.axis_index("s")
    zero16 = jnp.zeros((L,), _f32)

    def _z(j, _):
        for q in range(128 // L):
            deg_t[j, pl.ds(q * L, L)] = zero16
        return _
    lax.fori_loop(0, DROWS, _z, None)

    @pl.when(sid == 0)
    def _():
        pltpu.sync_copy(deg_t, deg_s)
    plsc.subcore_barrier()

    def _hist_chunk(ci, _):
        base = sid * EPS_SC + ci * CH
        pltpu.sync_copy(dst_hbm.at[pl.ds(base, CH)], dstb)
        pltpu.sync_copy(et_hbm.at[pl.ds(base, CH)], etb)

        def _grp(g, _):
            d16 = dstb[pl.ds(g * L, L)]
            e16 = etb[pl.ds(g * L, L)]
            idx = e16 * N + d16
            cnt, lastm = plsc.scan_count(idx)
            row = lax.shift_right_logical(idx, 7)
            col = jnp.bitwise_and(idx, 127)
            plsc.addupdate_scatter(deg_t, [row, col], cnt.astype(_f32),
                                   mask=lastm)
            return _
        lax.fori_loop(0, CH // L, _grp, None)
        return _
    lax.fori_loop(0, EPS_SC // CH, _hist_chunk, None)
    plsc.subcore_barrier()

    iota16 = lax.iota(_i32, L)
    for cc in range(3):
        for gg in range(8):
            idxrows[cc * 8, pl.ds(gg * L, L)] = iota16 + (cc * 128 + gg * L)
    for cc in range(3):
        pltpu.sync_copy(deg_t.at[pl.ds(cc * 128, 128)],
                        deg_s.at[idxrows.at[cc * 8]], add=True)
    plsc.subcore_barrier()

    pltpu.sync_copy(deg_s, deg_t)

    def _s(j, _):
        for q in range(128 // L):
            v = deg_t[j, pl.ds(q * L, L)]
            deg_t[j, pl.ds(q * L, L)] = 1.0 / jnp.maximum(v, 1.0)
        return _
    lax.fori_loop(0, DROWS, _s, None)

    gwid = cid * NS + sid

    def _out_chunk(ci, _):
        base = gwid * EPT + ci * CH
        pltpu.sync_copy(src_hbm.at[pl.ds(base, CH)], srcb)
        pltpu.sync_copy(dst_hbm.at[pl.ds(base, CH)], dstb)
        pltpu.sync_copy(et_hbm.at[pl.ds(base, CH)], etb)

        def _grp(g, _):
            s16 = srcb[pl.ds(g * L, L)]
            d16 = dstb[pl.ds(g * L, L)]
            e16 = etb[pl.ds(g * L, L)]
            gout[pl.ds(g * L, L)] = e16 * N + s16
            sidx = e16 * N + d16
            srow = lax.shift_right_logical(sidx, 7)
            scol = jnp.bitwise_and(sidx, 127)
            sout[pl.ds(g * L, L)] = plsc.load_gather(deg_t, [srow, scol])
            return _
        lax.fori_loop(0, CH // L, _grp, None)
        pltpu.sync_copy(gout, gsrc_hbm.at[pl.ds(base, CH)])
        pltpu.sync_copy(sout, sedge_hbm.at[pl.ds(base, CH)])
        return _
    lax.fori_loop(0, EPT // CH, _out_chunk, None)


def _make_layer(D):
    RCH = N // K
    NCH = CH // K
    NB = 3

    @functools.partial(
        pl.kernel,
        out_type=jax.ShapeDtypeStruct((2 * N, D), _f32),
        mesh=_mesh,
        compiler_params=pltpu.CompilerParams(needs_layout_passes=False),
        scratch_types=[
            pltpu.VMEM((CH,), _i32),
            pltpu.VMEM((CH,), _i32),
            pltpu.VMEM((CH,), _f32),
            [pltpu.VMEM((K,), _i32)] * NB,
            [pltpu.VMEM((K,), _i32)] * NB,
            [pltpu.VMEM((K, D), _f32)] * NB,
            [pltpu.SemaphoreType.DMA] * NB,
            [pltpu.SemaphoreType.DMA] * NB,
            pltpu.VMEM_SHARED((N, D), _f32),
        ],
    )
    def _layer(y_hbm, gsrc_hbm, dst_hbm, sed_hbm, m_hbm,
               gbuf, dbuf, sbuf, g80s, d80s, rowss, gsems, ssems, msh):
        cid = lax.axis_index("c")
        sid = lax.axis_index("s")
        zero16 = jnp.zeros((L,), _f32)

        rows0 = rowss[0]
        for j in range(K):
            for d in range(D // L):
                rows0[j, pl.ds(d * L, L)] = zero16
        nu = (RCH - sid + NS - 1) // NS

        def _z(k, _):
            u = sid + k * NS
            pltpu.sync_copy(rows0, msh.at[pl.ds(u * K, K)])
            return _
        lax.fori_loop(0, nu, _z, None)
        plsc.subcore_barrier()

        ebase = cid * (E // 2) + sid * EPT

        def _fire(i, sl):
            g80, d80 = g80s[sl], d80s[sl]
            for q in range(K // L):
                g80[pl.ds(q * L, L)] = gbuf[pl.ds(i * K + q * L, L)]
                d80[pl.ds(q * L, L)] = dbuf[pl.ds(i * K + q * L, L)]
            pltpu.async_copy(y_hbm.at[g80], rowss[sl], gsems[sl])

        def _process(i, sl):
            rows = rowss[sl]
            pltpu.make_async_copy(y_hbm.at[g80s[sl]], rows, gsems[sl]).wait()

            def _scale(g, _):
                s16 = sbuf[pl.ds(i * K + g * L, L)]
                for jj in range(L):
                    sv = jnp.full((L,), s16[jj], _f32)
                    j = g * L + jj
                    for d in range(D // L):
                        rows[j, pl.ds(d * L, L)] = rows[j, pl.ds(d * L, L)] * sv
                return _
            lax.fori_loop(0, K // L, _scale, None)
            pltpu.async_copy(rows, msh.at[d80s[sl]], ssems[sl], add=True)

        def _wait_scat(sl):
            pltpu.make_async_copy(rowss[sl], msh.at[d80s[sl]],
                                  ssems[sl]).wait()

        def _blk(t, _):
            base = ebase + t * CH
            pltpu.sync_copy(gsrc_hbm.at[pl.ds(base, CH)], gbuf)
            pltpu.sync_copy(dst_hbm.at[pl.ds(base, CH)], dbuf)
            pltpu.sync_copy(sed_hbm.at[pl.ds(base, CH)], sbuf)
            for sl in range(NB):
                _fire(sl, sl)

            def _trip(pp, _):
                i0 = NB * pp
                for off in range(NB):
                    i = i0 + off

                    @pl.when(i < NCH)
                    def _():
                        _process(i, off)
                for off in range(NB):
                    j = i0 + NB + off

                    @pl.when(j < NCH)
                    def _():
                        _wait_scat(off)
                        _fire(j, off)
                return _
            lax.fori_loop(0, (NCH + NB - 1) // NB, _trip, None)
            for sl in range(NB):
                _wait_scat(sl)
            return _
        lax.fori_loop(0, EPT // CH, _blk, None)
        plsc.subcore_barrier()

        def _wb(k, _):
            off = (sid + k * NS) * K
            pltpu.sync_copy(msh.at[pl.ds(off, K)],
                            m_hbm.at[pl.ds(cid * N + off, K)])
            return _
        lax.fori_loop(0, nu, _wb, None)

    return _layer


_layer128 = _make_layer(DH)


@functools.partial(
    pl.kernel,
    out_type=jax.ShapeDtypeStruct((E,), _f32),
    mesh=_mesh,
    compiler_params=pltpu.CompilerParams(needs_layout_passes=False),
    scratch_types=[
        pltpu.VMEM((CH,), _i32),
        pltpu.VMEM((CH,), _i32),
        pltpu.VMEM((K,), _i32),
        pltpu.VMEM((K,), _i32),
        pltpu.VMEM((K,), _i32),
        pltpu.VMEM((K,), _i32),
        pltpu.VMEM((K, DH), _f32),
        pltpu.VMEM((K, DH), _f32),
        pltpu.VMEM((K, DH), _f32),
        pltpu.VMEM((K, DH), _f32),
        pltpu.VMEM((CH,), _f32),
        pltpu.SemaphoreType.DMA,
        pltpu.SemaphoreType.DMA,
        pltpu.SemaphoreType.DMA,
        pltpu.SemaphoreType.DMA,
        pltpu.VMEM_SHARED((N, DH), _f32),
    ],
)
def _final(t_hbm, a_hbm, gsrc_hbm, dst_hbm, out_hbm,
           gbuf, dbuf, g80a, g80b, d80a, d80b, tra, trb, ara, arb, ob,
           semta, semtb, semaa, semab, ashr):
    cid = lax.axis_index("c")
    sid = lax.axis_index("s")
    gwid = cid * NS + sid
    ebase = gwid * EPT
    NCH = CH // K

    nu = ((N // K) - sid + NS - 1) // NS

    def _st(k, _):
        off = (sid + k * NS) * K
        pltpu.sync_copy(a_hbm.at[pl.ds(off, K)], ashr.at[pl.ds(off, K)])
        return _
    lax.fori_loop(0, nu, _st, None)
    plsc.subcore_barrier()

    def _fire(i, g80, d80, tr, ar, semt, sema):
        for q in range(K // L):
            g80[pl.ds(q * L, L)] = gbuf[pl.ds(i * K + q * L, L)]
            d80[pl.ds(q * L, L)] = dbuf[pl.ds(i * K + q * L, L)]
        pltpu.async_copy(t_hbm.at[g80], tr, semt)
        pltpu.async_copy(ashr.at[d80], ar, sema)

    def _process(i, g80, d80, tr, ar, semt, sema):
        pltpu.make_async_copy(t_hbm.at[g80], tr, semt).wait()
        pltpu.make_async_copy(ashr.at[d80], ar, sema).wait()
        lanes = lax.iota(_i32, L)

        def _dot(g, _):
            outv = jnp.zeros((L,), _f32)
            for jj in range(L):
                j = g * L + jj
                pr = tr[j, pl.ds(0, L)] * ar[j, pl.ds(0, L)]
                for d in range(1, C // L):
                    pr = pr + tr[j, pl.ds(d * L, L)] * ar[j, pl.ds(d * L, L)]
                outv = jnp.where(lanes == jj, jnp.sum(pr), outv)
            ob[pl.ds(i * K + g * L, L)] = outv
            return _
        lax.fori_loop(0, K // L, _dot, None)

    def _blk(t, _):
        base = ebase + t * CH
        pltpu.sync_copy(gsrc_hbm.at[pl.ds(base, CH)], gbuf)
        pltpu.sync_copy(dst_hbm.at[pl.ds(base, CH)], dbuf)
        _fire(0, g80a, d80a, tra, ara, semta, semaa)
        _fire(1, g80b, d80b, trb, arb, semtb, semab)

        def _pair(pp, _):
            ia = 2 * pp
            _process(ia, g80a, d80a, tra, ara, semta, semaa)

            @pl.when(ia + 2 < NCH)
            def _():
                _fire(ia + 2, g80a, d80a, tra, ara, semta, semaa)

            @pl.when(ia + 1 < NCH)
            def _():
                _process(ia + 1, g80b, d80b, trb, arb, semtb, semab)

                @pl.when(ia + 3 < NCH)
                def _():
                    _fire(ia + 3, g80b, d80b, trb, arb, semtb, semab)
            return _
        lax.fori_loop(0, (NCH + 1) // 2, _pair, None)
        pltpu.sync_copy(ob, out_hbm.at[pl.ds(base, CH)])
        return _
    lax.fori_loop(0, EPT // CH, _blk, None)


def kernel(x, edge_index, edge_type, W_rel1, W_self1, b1, W_rel2, W_self2,
           b2, inter_logits, gate_log_alpha, absent_bias):
    src = edge_index[0].astype(_i32)
    dst = edge_index[1].astype(_i32)
    et = edge_type.astype(_i32)

    Y1, S1 = _enc1(x, W_rel1, W_self1, b1.reshape(1, DH))
    gsrc, sedge = _prep(src, dst, et)

    M1 = _layer128(Y1.reshape(R * N, DH), gsrc, dst, sedge)
    Y2, S2 = _enc2(S1, M1.reshape(2, N, DH), W_rel2, W_self2, b2.reshape(1, C))

    M2 = _layer128(Y2.reshape(R * N, DH), gsrc, dst, sedge)
    A, T = _enc3(S2, M2.reshape(2, N, DH), inter_logits, gate_log_alpha,
                 absent_bias.reshape(1, R))

    return _final(T.reshape(R * N, DH), A, gsrc, dst)
